# Initial kernel scaffold; baseline (speedup 1.0000x reference)
import jax
import jax.numpy as jnp
from jax.experimental import pallas as pl


def kernel(x, edge_index, edge_attr, ew1, eb1, ew2, eb2, n1w1, n1b1, n1w2, n1b2, n2w1, n2b1, n2w2, n2b2):
    raise NotImplementedError("write your pallas kernel here")


# speedup vs baseline: 2.8574x; 2.8574x over previous
import functools

import jax
import jax.numpy as jnp
from jax import lax
from jax.experimental import pallas as pl
from jax.experimental.pallas import tpu as pltpu
from jax.experimental.pallas import tpu_sc as plsc

N = 10000
E = 160000
D = 256

NC = 2
NS = 16
NW = NC * NS
EPW = E // NW
CH = 40
NCH = EPW // CH
RPT = 624
TAIL = N - NS * RPT
DC = D // NC

_BN = 2000
_BE = 2000

EPT = E // NS
NCHT = EPT // CH


def _pre_body(x_ref, wr_ref, br_ref, wc_ref, tr_ref, tc_ref):
    x = x_ref[...]
    tr_ref[...] = x @ wr_ref[...] + br_ref[...]
    tc_ref[...] = x @ wc_ref[...]


def _pre(x, wrow, brow, wcol):
    return pl.pallas_call(
        _pre_body,
        grid=(N // _BN,),
        in_specs=[
            pl.BlockSpec((_BN, D), lambda i: (i, 0)),
            pl.BlockSpec((D, 2 * D), lambda i: (0, 0)),
            pl.BlockSpec((1, 2 * D), lambda i: (0, 0)),
            pl.BlockSpec((D, D), lambda i: (0, 0)),
        ],
        out_specs=[
            pl.BlockSpec((_BN, 2 * D), lambda i: (i, 0)),
            pl.BlockSpec((_BN, D), lambda i: (i, 0)),
        ],
        out_shape=[
            jax.ShapeDtypeStruct((N, 2 * D), jnp.float32),
            jax.ShapeDtypeStruct((N, D), jnp.float32),
        ],
    )(x, wrow, brow, wcol)


def _edge_body(gs_ref, gd_ref, ea_ref, wea_ref, ew2_ref, eb2_ref,
               wen_ref, n1w2_ref, n1b2_ref, en_ref, msg_ref):
    h1 = jnp.maximum(gs_ref[:, :D] + gd_ref[...] + ea_ref[...] @ wea_ref[...], 0.0)
    en = h1 @ ew2_ref[...] + eb2_ref[...]
    en_ref[...] = en
    h2 = jnp.maximum(gs_ref[:, D:] + en @ wen_ref[...], 0.0)
    msg_ref[...] = h2 @ n1w2_ref[...] + n1b2_ref[...]


def _edge(gsrc, gdst, ea, wea, ew2, eb2, wen, n1w2, n1b2):
    wspec = pl.BlockSpec((D, D), lambda i: (0, 0))
    bspec = pl.BlockSpec((1, D), lambda i: (0, 0))
    return pl.pallas_call(
        _edge_body,
        grid=(E // _BE,),
        in_specs=[
            pl.BlockSpec((_BE, 2 * D), lambda i: (i, 0)),
            pl.BlockSpec((_BE, D), lambda i: (i, 0)),
            pl.BlockSpec((_BE, D), lambda i: (i, 0)),
            wspec, wspec, bspec, wspec, wspec, bspec,
        ],
        out_specs=[
            pl.BlockSpec((_BE, D), lambda i: (i, 0)),
            pl.BlockSpec((_BE, D), lambda i: (i, 0)),
        ],
        out_shape=[
            jax.ShapeDtypeStruct((E, D), jnp.float32),
            jax.ShapeDtypeStruct((E, D), jnp.float32),
        ],
    )(gsrc, gdst, ea, wea, ew2, eb2, wen, n1w2, n1b2)


def _node_body(x_ref, agg_ref, ca_ref, cb_ref, w2a_ref, w2b_ref, b1_ref,
               w2_ref, b2_ref, out_ref):
    cnt = ca_ref[:, :1] + cb_ref[:, :1]
    mean = agg_ref[...] / jnp.maximum(cnt, 1.0)
    h = jnp.maximum(
        x_ref[...] @ w2a_ref[...] + mean @ w2b_ref[...] + b1_ref[...], 0.0)
    out_ref[...] = h @ w2_ref[...] + b2_ref[...]


def _node(x, agg, cnta, cntb, w2a, w2b, b1, w2, b2):
    wspec = pl.BlockSpec((D, D), lambda i: (0, 0))
    bspec = pl.BlockSpec((1, D), lambda i: (0, 0))
    return pl.pallas_call(
        _node_body,
        grid=(N // _BN,),
        in_specs=[
            pl.BlockSpec((_BN, D), lambda i: (i, 0)),
            pl.BlockSpec((_BN, D), lambda i: (i, 0)),
            pl.BlockSpec((_BN, DC), lambda i: (i, 0)),
            pl.BlockSpec((_BN, DC), lambda i: (i, 0)),
            wspec, wspec, bspec, wspec, bspec,
        ],
        out_specs=pl.BlockSpec((_BN, D), lambda i: (i, 0)),
        out_shape=jax.ShapeDtypeStruct((N, D), jnp.float32),
    )(x, agg, cnta, cntb, w2a, w2b, b1, w2, b2)


def _sc_gather(tabrow, tabcol, row, col):
    mesh = plsc.VectorSubcoreMesh(core_axis_name="c", subcore_axis_name="s")

    @functools.partial(
        pl.kernel,
        out_type=[
            jax.ShapeDtypeStruct((E, 2 * D), jnp.float32),
            jax.ShapeDtypeStruct((E, D), jnp.float32),
        ],
        mesh=mesh,
        scratch_types=[
            pltpu.VMEM((EPW,), jnp.int32),
            pltpu.VMEM((EPW,), jnp.int32),
            pltpu.VMEM((CH, 2 * D), jnp.float32),
            pltpu.VMEM((CH, 2 * D), jnp.float32),
            pltpu.VMEM((CH, D), jnp.float32),
            pltpu.VMEM((CH, D), jnp.float32),
            pltpu.SemaphoreType.DMA,
            pltpu.SemaphoreType.DMA,
            pltpu.SemaphoreType.DMA,
            pltpu.SemaphoreType.DMA,
        ],
    )
    def k(tr_hbm, tc_hbm, row_hbm, col_hbm, gs_hbm, gd_hbm,
          rowv, colv, a0, a1, b0, b1, sa0, sa1, sb0, sb1):
        wid = lax.axis_index("s") * NC + lax.axis_index("c")
        base = wid * EPW
        pltpu.sync_copy(row_hbm.at[pl.ds(base, EPW)], rowv)
        pltpu.sync_copy(col_hbm.at[pl.ds(base, EPW)], colv)

        abuf = (a0, a1)
        bbuf = (b0, b1)
        asem = (sa0, sa1)
        bsem = (sb0, sb1)

        def fire(j, p):
            off = j * CH
            pltpu.make_async_copy(
                tr_hbm.at[rowv.at[pl.ds(off, CH)]], abuf[p], asem[p]).start()
            pltpu.make_async_copy(
                tc_hbm.at[colv.at[pl.ds(off, CH)]], bbuf[p], bsem[p]).start()

        def drain_out(j, p):
            off = j * CH
            pltpu.make_async_copy(
                tr_hbm.at[rowv.at[pl.ds(off, CH)]], abuf[p], asem[p]).wait()
            pltpu.sync_copy(abuf[p], gs_hbm.at[pl.ds(base + off, CH)])
            pltpu.make_async_copy(
                tc_hbm.at[colv.at[pl.ds(off, CH)]], bbuf[p], bsem[p]).wait()
            pltpu.sync_copy(bbuf[p], gd_hbm.at[pl.ds(base + off, CH)])

        fire(0, 0)

        def body(i, carry):
            j = 2 * i
            fire(j + 1, 1)
            drain_out(j, 0)
            fire(j + 2, 0)
            drain_out(j + 1, 1)
            return carry

        lax.fori_loop(0, (NCH - 1) // 2, body, 0)
        drain_out(NCH - 1, 0)

    return k(tabrow, tabcol, row, col)


def _sc_scatter(msg, col):
    mesh = plsc.VectorSubcoreMesh(core_axis_name="c", subcore_axis_name="s")

    @functools.partial(
        pl.kernel,
        out_type=jax.ShapeDtypeStruct((N, D), jnp.float32),
        mesh=mesh,
        scratch_types=[
            pltpu.VMEM((CH,), jnp.int32),
            pltpu.VMEM((CH,), jnp.int32),
            pltpu.VMEM((CH, DC), jnp.float32),
            pltpu.VMEM((CH, DC), jnp.float32),
            pltpu.VMEM((16, DC), jnp.float32),
            pltpu.VMEM_SHARED((N, DC), jnp.float32),
            pltpu.SemaphoreType.DMA,
            pltpu.SemaphoreType.DMA,
            pltpu.SemaphoreType.DMA,
            pltpu.SemaphoreType.DMA,
        ],
    )
    def k(msg_hbm, col_hbm, agg_hbm,
          c0, c1, m0, m1, zt, aggsh, s0, s1, si0, si1):
        c = lax.axis_index("c")
        s = lax.axis_index("s")
        coff = c * DC
        base = s * EPT
        r0 = s * RPT

        for r in range(16):
            for q in range(DC // 16):
                zt[r, pl.ds(q * 16, 16)] = jnp.zeros((16,), jnp.float32)

        def zb(i, carry):
            pltpu.sync_copy(zt, aggsh.at[pl.ds(r0 + i * 16, 16)])
            return carry

        lax.fori_loop(0, RPT // 16, zb, 0)

        @pl.when(s == 0)
        def _():
            pltpu.sync_copy(zt, aggsh.at[pl.ds(NS * RPT, TAIL)])

        plsc.subcore_barrier()

        mbuf = (m0, m1)
        msem = (s0, s1)
        cbuf = (c0, c1)
        csem = (si0, si1)

        def fire(j, p):
            pltpu.make_async_copy(
                col_hbm.at[pl.ds(base + j * CH, CH)], cbuf[p], csem[p]).start()
            pltpu.make_async_copy(
                msg_hbm.at[pl.ds(base + j * CH, CH), pl.ds(coff, DC)],
                mbuf[p], msem[p]).start()

        def drain_scat(j, p):
            pltpu.make_async_copy(
                col_hbm.at[pl.ds(base + j * CH, CH)], cbuf[p], csem[p]).wait()
            pltpu.make_async_copy(
                msg_hbm.at[pl.ds(base + j * CH, CH), pl.ds(coff, DC)],
                mbuf[p], msem[p]).wait()
            pltpu.sync_copy(mbuf[p], aggsh.at[cbuf[p]], add=True)

        fire(0, 0)

        def body(i, carry):
            j = 2 * i
            fire(j + 1, 1)
            drain_scat(j, 0)
            fire(j + 2, 0)
            drain_scat(j + 1, 1)
            return carry

        lax.fori_loop(0, (NCHT - 2) // 2, body, 0)
        fire(NCHT - 1, 1)
        drain_scat(NCHT - 2, 0)
        drain_scat(NCHT - 1, 1)

        plsc.subcore_barrier()

        pltpu.sync_copy(aggsh.at[pl.ds(r0, RPT)],
                        agg_hbm.at[pl.ds(r0, RPT), pl.ds(coff, DC)])

        @pl.when(s == 0)
        def _():
            pltpu.sync_copy(aggsh.at[pl.ds(NS * RPT, TAIL)],
                            agg_hbm.at[pl.ds(NS * RPT, TAIL), pl.ds(coff, DC)])

    return k(msg, col)


def _sc_count(col):
    mesh = plsc.VectorSubcoreMesh(core_axis_name="c", subcore_axis_name="s")

    @functools.partial(
        pl.kernel,
        out_type=[
            jax.ShapeDtypeStruct((N, DC), jnp.float32),
            jax.ShapeDtypeStruct((N, DC), jnp.float32),
        ],
        mesh=mesh,
        scratch_types=[
            pltpu.VMEM((CH,), jnp.int32),
            pltpu.VMEM((CH,), jnp.int32),
            pltpu.VMEM((CH, DC), jnp.float32),
            pltpu.VMEM((16, DC), jnp.float32),
            pltpu.VMEM_SHARED((N, DC), jnp.float32),
            pltpu.SemaphoreType.DMA,
            pltpu.SemaphoreType.DMA,
        ],
    )
    def k(col_hbm, ca_hbm, cb_hbm, c0, c1, ones, zt, cntsh, si0, si1):
        c = lax.axis_index("c")
        s = lax.axis_index("s")
        wid = s * NC + c
        base = wid * EPW
        r0 = s * RPT

        for r in range(16):
            for q in range(DC // 16):
                zt[r, pl.ds(q * 16, 16)] = jnp.zeros((16,), jnp.float32)
        for r in range(CH):
            for q in range(DC // 16):
                ones[r, pl.ds(q * 16, 16)] = jnp.ones((16,), jnp.float32)

        def zb(i, carry):
            pltpu.sync_copy(zt, cntsh.at[pl.ds(r0 + i * 16, 16)])
            return carry

        lax.fori_loop(0, RPT // 16, zb, 0)

        @pl.when(s == 0)
        def _():
            pltpu.sync_copy(zt, cntsh.at[pl.ds(NS * RPT, TAIL)])

        plsc.subcore_barrier()

        cbuf = (c0, c1)
        csem = (si0, si1)

        def fire(j, p):
            pltpu.make_async_copy(
                col_hbm.at[pl.ds(base + j * CH, CH)], cbuf[p], csem[p]).start()

        def drain_scat(j, p):
            pltpu.make_async_copy(
                col_hbm.at[pl.ds(base + j * CH, CH)], cbuf[p], csem[p]).wait()
            pltpu.sync_copy(ones, cntsh.at[cbuf[p]], add=True)

        fire(0, 0)

        def body(i, carry):
            j = 2 * i
            fire(j + 1, 1)
            drain_scat(j, 0)
            fire(j + 2, 0)
            drain_scat(j + 1, 1)
            return carry

        lax.fori_loop(0, (NCH - 1) // 2, body, 0)
        drain_scat(NCH - 1, 0)

        plsc.subcore_barrier()

        @pl.when(c == 0)
        def _():
            pltpu.sync_copy(cntsh.at[pl.ds(r0, RPT)],
                            ca_hbm.at[pl.ds(r0, RPT)])

            @pl.when(s == 0)
            def _():
                pltpu.sync_copy(cntsh.at[pl.ds(NS * RPT, TAIL)],
                                ca_hbm.at[pl.ds(NS * RPT, TAIL)])

        @pl.when(c == 1)
        def _():
            pltpu.sync_copy(cntsh.at[pl.ds(r0, RPT)],
                            cb_hbm.at[pl.ds(r0, RPT)])

            @pl.when(s == 0)
            def _():
                pltpu.sync_copy(cntsh.at[pl.ds(NS * RPT, TAIL)],
                                cb_hbm.at[pl.ds(NS * RPT, TAIL)])

    return k(col)


def kernel(x, edge_index, edge_attr, ew1, eb1, ew2, eb2,
           n1w1, n1b1, n1w2, n1b2, n2w1, n2b1, n2w2, n2b2):
    row = edge_index[0]
    col = edge_index[1]

    wrow = jnp.concatenate([ew1[:D], n1w1[:D]], axis=1)
    brow = jnp.concatenate([eb1, n1b1])[None, :]
    wcol = ew1[D:2 * D]

    tabrow, tabcol = _pre(x, wrow, brow, wcol)
    gsrc, gdst = _sc_gather(tabrow, tabcol, row, col)
    e_new, msg = _edge(gsrc, gdst, edge_attr, ew1[2 * D:], ew2, eb2[None, :],
                       n1w1[D:], n1w2, n1b2[None, :])
    agg = _sc_scatter(msg, col)
    cnta, cntb = _sc_count(col)
    x_new = _node(x, agg, cnta, cntb, n2w1[:D], n2w1[D:], n2b1[None, :],
                  n2w2, n2b2[None, :])
    return (x_new, e_new)


# speedup vs baseline: 3.6212x; 1.2673x over previous
import functools

import jax
import jax.numpy as jnp
from jax import lax
from jax.experimental import pallas as pl
from jax.experimental.pallas import tpu as pltpu
from jax.experimental.pallas import tpu_sc as plsc

N = 10000
E = 160000
D = 256

NC = 2
NS = 16
NW = NC * NS
EPW = E // NW
CH = 40
NCH = EPW // CH
RPT = 624
TAIL = N - NS * RPT
DC = D // NC

_BN = 2000
_BE = 2000

EPT = E // NS
NCHT = EPT // CH


def _pack2(lo, hi):
    lo16 = jax.lax.bitcast_convert_type(lo.astype(jnp.bfloat16), jnp.uint16)
    hi16 = jax.lax.bitcast_convert_type(hi.astype(jnp.bfloat16), jnp.uint16)
    return lo16.astype(jnp.int32) | (hi16.astype(jnp.int32) << 16)


def _unpack2(w):
    wu = jax.lax.bitcast_convert_type(w, jnp.uint32)
    lo = jax.lax.bitcast_convert_type(
        (wu & 0xFFFF).astype(jnp.uint16), jnp.bfloat16)
    hi = jax.lax.bitcast_convert_type(
        (wu >> 16).astype(jnp.uint16), jnp.bfloat16)
    return lo.astype(jnp.float32), hi.astype(jnp.float32)


def _pre_body(x_ref, wr_ref, br_ref, wc_ref, tr_ref, tc_ref):
    x = x_ref[...]
    tr = x @ wr_ref[...] + br_ref[...]
    tr_ref[...] = _pack2(tr[:, :D], tr[:, D:])
    tc = x @ wc_ref[...]
    tc_ref[...] = _pack2(tc[:, :DC], tc[:, DC:])


def _pre(x, wrow, brow, wcol):
    return pl.pallas_call(
        _pre_body,
        grid=(N // _BN,),
        in_specs=[
            pl.BlockSpec((_BN, D), lambda i: (i, 0)),
            pl.BlockSpec((D, 2 * D), lambda i: (0, 0)),
            pl.BlockSpec((1, 2 * D), lambda i: (0, 0)),
            pl.BlockSpec((D, D), lambda i: (0, 0)),
        ],
        out_specs=[
            pl.BlockSpec((_BN, D), lambda i: (i, 0)),
            pl.BlockSpec((_BN, DC), lambda i: (i, 0)),
        ],
        out_shape=[
            jax.ShapeDtypeStruct((N, D), jnp.int32),
            jax.ShapeDtypeStruct((N, DC), jnp.int32),
        ],
    )(x, wrow, brow, wcol)


def _edge_body(gs_ref, gd_ref, ea_ref, wea_ref, ew2_ref, eb2_ref,
               wen_ref, n1w2_ref, n1b2_ref, en_ref, msg_ref):
    a, m = _unpack2(gs_ref[...])
    gdlo, gdhi = _unpack2(gd_ref[...])
    gd = jnp.concatenate([gdlo, gdhi], axis=1)
    f32 = jnp.float32
    bf = jnp.bfloat16
    h1 = jnp.maximum(
        a + gd + jnp.dot(ea_ref[...], wea_ref[...], preferred_element_type=f32),
        0.0)
    en = jnp.dot(h1.astype(bf), ew2_ref[...], preferred_element_type=f32)
    en = en + eb2_ref[...]
    en_ref[...] = en
    h2 = jnp.maximum(
        m + jnp.dot(en.astype(bf), wen_ref[...], preferred_element_type=f32),
        0.0)
    msg_ref[...] = jnp.dot(
        h2.astype(bf), n1w2_ref[...], preferred_element_type=f32) + n1b2_ref[...]


def _edge(gsrc, gdst, ea, wea, ew2, eb2, wen, n1w2, n1b2):
    wspec = pl.BlockSpec((D, D), lambda i: (0, 0))
    bspec = pl.BlockSpec((1, D), lambda i: (0, 0))
    return pl.pallas_call(
        _edge_body,
        grid=(E // _BE,),
        in_specs=[
            pl.BlockSpec((_BE, D), lambda i: (i, 0)),
            pl.BlockSpec((_BE, DC), lambda i: (i, 0)),
            pl.BlockSpec((_BE, D), lambda i: (i, 0)),
            wspec, wspec, bspec, wspec, wspec, bspec,
        ],
        out_specs=[
            pl.BlockSpec((_BE, D), lambda i: (i, 0)),
            pl.BlockSpec((_BE, D), lambda i: (i, 0)),
        ],
        out_shape=[
            jax.ShapeDtypeStruct((E, D), jnp.float32),
            jax.ShapeDtypeStruct((E, D), jnp.float32),
        ],
    )(gsrc, gdst, ea, wea, ew2, eb2, wen, n1w2, n1b2)


def _node_body(x_ref, agg_ref, ca_ref, cb_ref, w2a_ref, w2b_ref, b1_ref,
               w2_ref, b2_ref, out_ref):
    cnt = ca_ref[:, :1] + cb_ref[:, :1]
    mean = agg_ref[...] / jnp.maximum(cnt, 1.0)
    h = jnp.maximum(
        x_ref[...] @ w2a_ref[...] + mean @ w2b_ref[...] + b1_ref[...], 0.0)
    out_ref[...] = h @ w2_ref[...] + b2_ref[...]


def _node(x, agg, cnta, cntb, w2a, w2b, b1, w2, b2):
    wspec = pl.BlockSpec((D, D), lambda i: (0, 0))
    bspec = pl.BlockSpec((1, D), lambda i: (0, 0))
    return pl.pallas_call(
        _node_body,
        grid=(N // _BN,),
        in_specs=[
            pl.BlockSpec((_BN, D), lambda i: (i, 0)),
            pl.BlockSpec((_BN, D), lambda i: (i, 0)),
            pl.BlockSpec((_BN, DC), lambda i: (i, 0)),
            pl.BlockSpec((_BN, DC), lambda i: (i, 0)),
            wspec, wspec, bspec, wspec, bspec,
        ],
        out_specs=pl.BlockSpec((_BN, D), lambda i: (i, 0)),
        out_shape=jax.ShapeDtypeStruct((N, D), jnp.float32),
    )(x, agg, cnta, cntb, w2a, w2b, b1, w2, b2)


def _sc_gather(tabrow, tabcol, row, col):
    mesh = plsc.VectorSubcoreMesh(core_axis_name="c", subcore_axis_name="s")

    @functools.partial(
        pl.kernel,
        out_type=[
            jax.ShapeDtypeStruct((E, D), jnp.int32),
            jax.ShapeDtypeStruct((E, DC), jnp.int32),
        ],
        mesh=mesh,
        scratch_types=[
            pltpu.VMEM((EPW,), jnp.int32),
            pltpu.VMEM((EPW,), jnp.int32),
            pltpu.VMEM((CH, D), jnp.int32),
            pltpu.VMEM((CH, D), jnp.int32),
            pltpu.VMEM((CH, DC), jnp.int32),
            pltpu.VMEM((CH, DC), jnp.int32),
            pltpu.SemaphoreType.DMA,
            pltpu.SemaphoreType.DMA,
            pltpu.SemaphoreType.DMA,
            pltpu.SemaphoreType.DMA,
        ],
    )
    def k(tr_hbm, tc_hbm, row_hbm, col_hbm, gs_hbm, gd_hbm,
          rowv, colv, a0, a1, b0, b1, sa0, sa1, sb0, sb1):
        wid = lax.axis_index("s") * NC + lax.axis_index("c")
        base = wid * EPW
        pltpu.sync_copy(row_hbm.at[pl.ds(base, EPW)], rowv)
        pltpu.sync_copy(col_hbm.at[pl.ds(base, EPW)], colv)

        abuf = (a0, a1)
        bbuf = (b0, b1)
        asem = (sa0, sa1)
        bsem = (sb0, sb1)

        def fire(j, p):
            off = j * CH
            pltpu.make_async_copy(
                tr_hbm.at[rowv.at[pl.ds(off, CH)]], abuf[p], asem[p]).start()
            pltpu.make_async_copy(
                tc_hbm.at[colv.at[pl.ds(off, CH)]], bbuf[p], bsem[p]).start()

        def drain_out(j, p):
            off = j * CH
            pltpu.make_async_copy(
                tr_hbm.at[rowv.at[pl.ds(off, CH)]], abuf[p], asem[p]).wait()
            pltpu.sync_copy(abuf[p], gs_hbm.at[pl.ds(base + off, CH)])
            pltpu.make_async_copy(
                tc_hbm.at[colv.at[pl.ds(off, CH)]], bbuf[p], bsem[p]).wait()
            pltpu.sync_copy(bbuf[p], gd_hbm.at[pl.ds(base + off, CH)])

        fire(0, 0)

        def body(i, carry):
            j = 2 * i
            fire(j + 1, 1)
            drain_out(j, 0)
            fire(j + 2, 0)
            drain_out(j + 1, 1)
            return carry

        lax.fori_loop(0, (NCH - 1) // 2, body, 0)
        drain_out(NCH - 1, 0)

    return k(tabrow, tabcol, row, col)


def _sc_scatter(msg, col):
    mesh = plsc.VectorSubcoreMesh(core_axis_name="c", subcore_axis_name="s")

    @functools.partial(
        pl.kernel,
        out_type=jax.ShapeDtypeStruct((N, D), jnp.float32),
        mesh=mesh,
        scratch_types=[
            pltpu.VMEM((CH,), jnp.int32),
            pltpu.VMEM((CH,), jnp.int32),
            pltpu.VMEM((CH, DC), jnp.float32),
            pltpu.VMEM((CH, DC), jnp.float32),
            pltpu.VMEM((16, DC), jnp.float32),
            pltpu.VMEM_SHARED((N, DC), jnp.float32),
            pltpu.SemaphoreType.DMA,
            pltpu.SemaphoreType.DMA,
            pltpu.SemaphoreType.DMA,
            pltpu.SemaphoreType.DMA,
        ],
    )
    def k(msg_hbm, col_hbm, agg_hbm,
          c0, c1, m0, m1, zt, aggsh, s0, s1, si0, si1):
        c = lax.axis_index("c")
        s = lax.axis_index("s")
        coff = c * DC
        base = s * EPT
        r0 = s * RPT

        for r in range(16):
            for q in range(DC // 16):
                zt[r, pl.ds(q * 16, 16)] = jnp.zeros((16,), jnp.float32)

        def zb(i, carry):
            pltpu.sync_copy(zt, aggsh.at[pl.ds(r0 + i * 16, 16)])
            return carry

        lax.fori_loop(0, RPT // 16, zb, 0)

        @pl.when(s == 0)
        def _():
            pltpu.sync_copy(zt, aggsh.at[pl.ds(NS * RPT, TAIL)])

        plsc.subcore_barrier()

        mbuf = (m0, m1)
        msem = (s0, s1)
        cbuf = (c0, c1)
        csem = (si0, si1)

        def fire(j, p):
            pltpu.make_async_copy(
                col_hbm.at[pl.ds(base + j * CH, CH)], cbuf[p], csem[p]).start()
            pltpu.make_async_copy(
                msg_hbm.at[pl.ds(base + j * CH, CH), pl.ds(coff, DC)],
                mbuf[p], msem[p]).start()

        def drain_scat(j, p):
            pltpu.make_async_copy(
                col_hbm.at[pl.ds(base + j * CH, CH)], cbuf[p], csem[p]).wait()
            pltpu.make_async_copy(
                msg_hbm.at[pl.ds(base + j * CH, CH), pl.ds(coff, DC)],
                mbuf[p], msem[p]).wait()
            pltpu.sync_copy(mbuf[p], aggsh.at[cbuf[p]], add=True)

        fire(0, 0)

        def body(i, carry):
            j = 2 * i
            fire(j + 1, 1)
            drain_scat(j, 0)
            fire(j + 2, 0)
            drain_scat(j + 1, 1)
            return carry

        lax.fori_loop(0, (NCHT - 2) // 2, body, 0)
        fire(NCHT - 1, 1)
        drain_scat(NCHT - 2, 0)
        drain_scat(NCHT - 1, 1)

        plsc.subcore_barrier()

        pltpu.sync_copy(aggsh.at[pl.ds(r0, RPT)],
                        agg_hbm.at[pl.ds(r0, RPT), pl.ds(coff, DC)])

        @pl.when(s == 0)
        def _():
            pltpu.sync_copy(aggsh.at[pl.ds(NS * RPT, TAIL)],
                            agg_hbm.at[pl.ds(NS * RPT, TAIL), pl.ds(coff, DC)])

    return k(msg, col)


def _sc_count(col):
    mesh = plsc.VectorSubcoreMesh(core_axis_name="c", subcore_axis_name="s")

    @functools.partial(
        pl.kernel,
        out_type=[
            jax.ShapeDtypeStruct((N, DC), jnp.float32),
            jax.ShapeDtypeStruct((N, DC), jnp.float32),
        ],
        mesh=mesh,
        scratch_types=[
            pltpu.VMEM((CH,), jnp.int32),
            pltpu.VMEM((CH,), jnp.int32),
            pltpu.VMEM((CH, DC), jnp.float32),
            pltpu.VMEM((16, DC), jnp.float32),
            pltpu.VMEM_SHARED((N, DC), jnp.float32),
            pltpu.SemaphoreType.DMA,
            pltpu.SemaphoreType.DMA,
        ],
    )
    def k(col_hbm, ca_hbm, cb_hbm, c0, c1, ones, zt, cntsh, si0, si1):
        c = lax.axis_index("c")
        s = lax.axis_index("s")
        wid = s * NC + c
        base = wid * EPW
        r0 = s * RPT

        for r in range(16):
            for q in range(DC // 16):
                zt[r, pl.ds(q * 16, 16)] = jnp.zeros((16,), jnp.float32)
        for r in range(CH):
            for q in range(DC // 16):
                ones[r, pl.ds(q * 16, 16)] = jnp.ones((16,), jnp.float32)

        def zb(i, carry):
            pltpu.sync_copy(zt, cntsh.at[pl.ds(r0 + i * 16, 16)])
            return carry

        lax.fori_loop(0, RPT // 16, zb, 0)

        @pl.when(s == 0)
        def _():
            pltpu.sync_copy(zt, cntsh.at[pl.ds(NS * RPT, TAIL)])

        plsc.subcore_barrier()

        cbuf = (c0, c1)
        csem = (si0, si1)

        def fire(j, p):
            pltpu.make_async_copy(
                col_hbm.at[pl.ds(base + j * CH, CH)], cbuf[p], csem[p]).start()

        def drain_scat(j, p):
            pltpu.make_async_copy(
                col_hbm.at[pl.ds(base + j * CH, CH)], cbuf[p], csem[p]).wait()
            pltpu.sync_copy(ones, cntsh.at[cbuf[p]], add=True)

        fire(0, 0)

        def body(i, carry):
            j = 2 * i
            fire(j + 1, 1)
            drain_scat(j, 0)
            fire(j + 2, 0)
            drain_scat(j + 1, 1)
            return carry

        lax.fori_loop(0, (NCH - 1) // 2, body, 0)
        drain_scat(NCH - 1, 0)

        plsc.subcore_barrier()

        @pl.when(c == 0)
        def _():
            pltpu.sync_copy(cntsh.at[pl.ds(r0, RPT)],
                            ca_hbm.at[pl.ds(r0, RPT)])

            @pl.when(s == 0)
            def _():
                pltpu.sync_copy(cntsh.at[pl.ds(NS * RPT, TAIL)],
                                ca_hbm.at[pl.ds(NS * RPT, TAIL)])

        @pl.when(c == 1)
        def _():
            pltpu.sync_copy(cntsh.at[pl.ds(r0, RPT)],
                            cb_hbm.at[pl.ds(r0, RPT)])

            @pl.when(s == 0)
            def _():
                pltpu.sync_copy(cntsh.at[pl.ds(NS * RPT, TAIL)],
                                cb_hbm.at[pl.ds(NS * RPT, TAIL)])

    return k(col)


def kernel(x, edge_index, edge_attr, ew1, eb1, ew2, eb2,
           n1w1, n1b1, n1w2, n1b2, n2w1, n2b1, n2w2, n2b2):
    row = edge_index[0]
    col = edge_index[1]

    wrow = jnp.concatenate([ew1[:D], n1w1[:D]], axis=1)
    brow = jnp.concatenate([eb1, n1b1])[None, :]
    wcol = ew1[D:2 * D]

    bf = jnp.bfloat16
    tabrow, tabcol = _pre(x, wrow, brow, wcol)
    gsrc, gdst = _sc_gather(tabrow, tabcol, row, col)
    e_new, msg = _edge(gsrc, gdst, edge_attr.astype(bf),
                       ew1[2 * D:].astype(bf), ew2.astype(bf), eb2[None, :],
                       n1w1[D:].astype(bf), n1w2.astype(bf), n1b2[None, :])
    agg = _sc_scatter(msg, col)
    cnta, cntb = _sc_count(col)
    x_new = _node(x, agg, cnta, cntb, n2w1[:D], n2w1[D:], n2b1[None, :],
                  n2w2, n2b2[None, :])
    return (x_new, e_new)


# speedup vs baseline: 3.8880x; 1.0737x over previous
import functools

import jax
import jax.numpy as jnp
from jax import lax
from jax.experimental import pallas as pl
from jax.experimental.pallas import tpu as pltpu
from jax.experimental.pallas import tpu_sc as plsc

N = 10000
E = 160000
D = 256

NC = 2
NS = 16
NW = NC * NS
EPW = E // NW
CH = 40
NCH = EPW // CH
RPT = 624
TAIL = N - NS * RPT
DC = D // NC

_BN = 2000
_BE = 2000

EPT = E // NS
NCHT = EPT // CH


def _pack2(lo, hi):
    lo16 = jax.lax.bitcast_convert_type(lo.astype(jnp.bfloat16), jnp.uint16)
    hi16 = jax.lax.bitcast_convert_type(hi.astype(jnp.bfloat16), jnp.uint16)
    return lo16.astype(jnp.int32) | (hi16.astype(jnp.int32) << 16)


def _unpack2(w):
    wu = jax.lax.bitcast_convert_type(w, jnp.uint32)
    lo = jax.lax.bitcast_convert_type(
        (wu & 0xFFFF).astype(jnp.uint16), jnp.bfloat16)
    hi = jax.lax.bitcast_convert_type(
        (wu >> 16).astype(jnp.uint16), jnp.bfloat16)
    return lo.astype(jnp.float32), hi.astype(jnp.float32)


def _pre_body(x_ref, wr_ref, br_ref, wc_ref, tr_ref, tc_ref):
    x = x_ref[...]
    tr = x @ wr_ref[...] + br_ref[...]
    tr_ref[...] = _pack2(tr[:, :D], tr[:, D:])
    tc = x @ wc_ref[...]
    tc_ref[...] = _pack2(tc[:, :DC], tc[:, DC:])


def _pre(x, wrow, brow, wcol):
    return pl.pallas_call(
        _pre_body,
        grid=(N // _BN,),
        in_specs=[
            pl.BlockSpec((_BN, D), lambda i: (i, 0)),
            pl.BlockSpec((D, 2 * D), lambda i: (0, 0)),
            pl.BlockSpec((1, 2 * D), lambda i: (0, 0)),
            pl.BlockSpec((D, D), lambda i: (0, 0)),
        ],
        out_specs=[
            pl.BlockSpec((_BN, D), lambda i: (i, 0)),
            pl.BlockSpec((_BN, DC), lambda i: (i, 0)),
        ],
        out_shape=[
            jax.ShapeDtypeStruct((N, D), jnp.int32),
            jax.ShapeDtypeStruct((N, DC), jnp.int32),
        ],
    )(x, wrow, brow, wcol)


def _edge_body(gs_ref, gd_ref, ea_ref, wea_ref, ew2_ref, eb2_ref,
               wen_ref, n1w2_ref, n1b2_ref, en_ref, msg_ref):
    a, m = _unpack2(gs_ref[...])
    gdlo, gdhi = _unpack2(gd_ref[...])
    gd = jnp.concatenate([gdlo, gdhi], axis=1)
    f32 = jnp.float32
    bf = jnp.bfloat16
    h1 = jnp.maximum(
        a + gd + jnp.dot(ea_ref[...].astype(bf), wea_ref[...],
                         preferred_element_type=f32),
        0.0)
    en = jnp.dot(h1.astype(bf), ew2_ref[...], preferred_element_type=f32)
    en = en + eb2_ref[...]
    en_ref[...] = en
    h2 = jnp.maximum(
        m + jnp.dot(en.astype(bf), wen_ref[...], preferred_element_type=f32),
        0.0)
    msg_ref[...] = jnp.dot(
        h2.astype(bf), n1w2_ref[...], preferred_element_type=f32) + n1b2_ref[...]


def _edge(gsrc, gdst, ea, wea, ew2, eb2, wen, n1w2, n1b2):
    wspec = pl.BlockSpec((D, D), lambda i: (0, 0))
    bspec = pl.BlockSpec((1, D), lambda i: (0, 0))
    return pl.pallas_call(
        _edge_body,
        grid=(E // _BE,),
        in_specs=[
            pl.BlockSpec((_BE, D), lambda i: (i, 0)),
            pl.BlockSpec((_BE, DC), lambda i: (i, 0)),
            pl.BlockSpec((_BE, D), lambda i: (i, 0)),
            wspec, wspec, bspec, wspec, wspec, bspec,
        ],
        out_specs=[
            pl.BlockSpec((_BE, D), lambda i: (i, 0)),
            pl.BlockSpec((_BE, D), lambda i: (i, 0)),
        ],
        out_shape=[
            jax.ShapeDtypeStruct((E, D), jnp.float32),
            jax.ShapeDtypeStruct((E, D), jnp.float32),
        ],
    )(gsrc, gdst, ea, wea, ew2, eb2, wen, n1w2, n1b2)


def _node_body(x_ref, agg_ref, ca_ref, cb_ref, w2a_ref, w2b_ref, b1_ref,
               w2_ref, b2_ref, out_ref):
    cnt = ca_ref[:, :1] + cb_ref[:, :1]
    mean = agg_ref[...] / jnp.maximum(cnt, 1.0)
    h = jnp.maximum(
        x_ref[...] @ w2a_ref[...] + mean @ w2b_ref[...] + b1_ref[...], 0.0)
    out_ref[...] = h @ w2_ref[...] + b2_ref[...]


def _node(x, agg, cnta, cntb, w2a, w2b, b1, w2, b2):
    wspec = pl.BlockSpec((D, D), lambda i: (0, 0))
    bspec = pl.BlockSpec((1, D), lambda i: (0, 0))
    return pl.pallas_call(
        _node_body,
        grid=(N // _BN,),
        in_specs=[
            pl.BlockSpec((_BN, D), lambda i: (i, 0)),
            pl.BlockSpec((_BN, D), lambda i: (i, 0)),
            pl.BlockSpec((_BN, DC), lambda i: (i, 0)),
            pl.BlockSpec((_BN, DC), lambda i: (i, 0)),
            wspec, wspec, bspec, wspec, bspec,
        ],
        out_specs=pl.BlockSpec((_BN, D), lambda i: (i, 0)),
        out_shape=jax.ShapeDtypeStruct((N, D), jnp.float32),
    )(x, agg, cnta, cntb, w2a, w2b, b1, w2, b2)


def _sc_gather(tabrow, tabcol, row, col):
    mesh = plsc.VectorSubcoreMesh(core_axis_name="c", subcore_axis_name="s")

    @functools.partial(
        pl.kernel,
        out_type=[
            jax.ShapeDtypeStruct((E, D), jnp.int32),
            jax.ShapeDtypeStruct((E, DC), jnp.int32),
        ],
        mesh=mesh,
        scratch_types=[
            pltpu.VMEM((EPW,), jnp.int32),
            pltpu.VMEM((EPW,), jnp.int32),
            pltpu.VMEM((CH, D), jnp.int32),
            pltpu.VMEM((CH, D), jnp.int32),
            pltpu.VMEM((CH, DC), jnp.int32),
            pltpu.VMEM((CH, DC), jnp.int32),
            pltpu.SemaphoreType.DMA,
            pltpu.SemaphoreType.DMA,
            pltpu.SemaphoreType.DMA,
            pltpu.SemaphoreType.DMA,
        ],
    )
    def k(tr_hbm, tc_hbm, row_hbm, col_hbm, gs_hbm, gd_hbm,
          rowv, colv, a0, a1, b0, b1, sa0, sa1, sb0, sb1):
        wid = lax.axis_index("s") * NC + lax.axis_index("c")
        base = wid * EPW
        pltpu.sync_copy(row_hbm.at[pl.ds(base, EPW)], rowv)
        pltpu.sync_copy(col_hbm.at[pl.ds(base, EPW)], colv)

        abuf = (a0, a1)
        bbuf = (b0, b1)
        asem = (sa0, sa1)
        bsem = (sb0, sb1)

        def fire(j, p):
            off = j * CH
            pltpu.make_async_copy(
                tr_hbm.at[rowv.at[pl.ds(off, CH)]], abuf[p], asem[p]).start()
            pltpu.make_async_copy(
                tc_hbm.at[colv.at[pl.ds(off, CH)]], bbuf[p], bsem[p]).start()

        def drain_out(j, p):
            off = j * CH
            pltpu.make_async_copy(
                tr_hbm.at[rowv.at[pl.ds(off, CH)]], abuf[p], asem[p]).wait()
            pltpu.sync_copy(abuf[p], gs_hbm.at[pl.ds(base + off, CH)])
            pltpu.make_async_copy(
                tc_hbm.at[colv.at[pl.ds(off, CH)]], bbuf[p], bsem[p]).wait()
            pltpu.sync_copy(bbuf[p], gd_hbm.at[pl.ds(base + off, CH)])

        fire(0, 0)

        def body(i, carry):
            j = 2 * i
            fire(j + 1, 1)
            drain_out(j, 0)
            fire(j + 2, 0)
            drain_out(j + 1, 1)
            return carry

        lax.fori_loop(0, (NCH - 1) // 2, body, 0)
        drain_out(NCH - 1, 0)

    return k(tabrow, tabcol, row, col)


def _sc_scatter(msg, col):
    mesh = plsc.VectorSubcoreMesh(core_axis_name="c", subcore_axis_name="s")

    @functools.partial(
        pl.kernel,
        out_type=jax.ShapeDtypeStruct((N, D), jnp.float32),
        mesh=mesh,
        scratch_types=[
            pltpu.VMEM((CH,), jnp.int32),
            pltpu.VMEM((CH,), jnp.int32),
            pltpu.VMEM((CH, DC), jnp.float32),
            pltpu.VMEM((CH, DC), jnp.float32),
            pltpu.VMEM((16, DC), jnp.float32),
            pltpu.VMEM_SHARED((N, DC), jnp.float32),
            pltpu.SemaphoreType.DMA,
            pltpu.SemaphoreType.DMA,
            pltpu.SemaphoreType.DMA,
            pltpu.SemaphoreType.DMA,
        ],
    )
    def k(msg_hbm, col_hbm, agg_hbm,
          c0, c1, m0, m1, zt, aggsh, s0, s1, si0, si1):
        c = lax.axis_index("c")
        s = lax.axis_index("s")
        coff = c * DC
        base = s * EPT
        r0 = s * RPT

        for r in range(16):
            for q in range(DC // 16):
                zt[r, pl.ds(q * 16, 16)] = jnp.zeros((16,), jnp.float32)

        def zb(i, carry):
            pltpu.sync_copy(zt, aggsh.at[pl.ds(r0 + i * 16, 16)])
            return carry

        lax.fori_loop(0, RPT // 16, zb, 0)

        @pl.when(s == 0)
        def _():
            pltpu.sync_copy(zt, aggsh.at[pl.ds(NS * RPT, TAIL)])

        plsc.subcore_barrier()

        mbuf = (m0, m1)
        msem = (s0, s1)
        cbuf = (c0, c1)
        csem = (si0, si1)

        def fire(j, p):
            pltpu.make_async_copy(
                col_hbm.at[pl.ds(base + j * CH, CH)], cbuf[p], csem[p]).start()
            pltpu.make_async_copy(
                msg_hbm.at[pl.ds(base + j * CH, CH), pl.ds(coff, DC)],
                mbuf[p], msem[p]).start()

        def drain_scat(j, p):
            pltpu.make_async_copy(
                col_hbm.at[pl.ds(base + j * CH, CH)], cbuf[p], csem[p]).wait()
            pltpu.make_async_copy(
                msg_hbm.at[pl.ds(base + j * CH, CH), pl.ds(coff, DC)],
                mbuf[p], msem[p]).wait()
            pltpu.sync_copy(mbuf[p], aggsh.at[cbuf[p]], add=True)

        fire(0, 0)

        def body(i, carry):
            j = 2 * i
            fire(j + 1, 1)
            drain_scat(j, 0)
            fire(j + 2, 0)
            drain_scat(j + 1, 1)
            return carry

        lax.fori_loop(0, (NCHT - 2) // 2, body, 0)
        fire(NCHT - 1, 1)
        drain_scat(NCHT - 2, 0)
        drain_scat(NCHT - 1, 1)

        plsc.subcore_barrier()

        pltpu.sync_copy(aggsh.at[pl.ds(r0, RPT)],
                        agg_hbm.at[pl.ds(r0, RPT), pl.ds(coff, DC)])

        @pl.when(s == 0)
        def _():
            pltpu.sync_copy(aggsh.at[pl.ds(NS * RPT, TAIL)],
                            agg_hbm.at[pl.ds(NS * RPT, TAIL), pl.ds(coff, DC)])

    return k(msg, col)


def _sc_count(col):
    mesh = plsc.VectorSubcoreMesh(core_axis_name="c", subcore_axis_name="s")

    @functools.partial(
        pl.kernel,
        out_type=[
            jax.ShapeDtypeStruct((N, DC), jnp.float32),
            jax.ShapeDtypeStruct((N, DC), jnp.float32),
        ],
        mesh=mesh,
        scratch_types=[
            pltpu.VMEM((CH,), jnp.int32),
            pltpu.VMEM((CH,), jnp.int32),
            pltpu.VMEM((CH, DC), jnp.float32),
            pltpu.VMEM((16, DC), jnp.float32),
            pltpu.VMEM_SHARED((N, DC), jnp.float32),
            pltpu.SemaphoreType.DMA,
            pltpu.SemaphoreType.DMA,
        ],
    )
    def k(col_hbm, ca_hbm, cb_hbm, c0, c1, ones, zt, cntsh, si0, si1):
        c = lax.axis_index("c")
        s = lax.axis_index("s")
        wid = s * NC + c
        base = wid * EPW
        r0 = s * RPT

        for r in range(16):
            for q in range(DC // 16):
                zt[r, pl.ds(q * 16, 16)] = jnp.zeros((16,), jnp.float32)
        for r in range(CH):
            for q in range(DC // 16):
                ones[r, pl.ds(q * 16, 16)] = jnp.ones((16,), jnp.float32)

        def zb(i, carry):
            pltpu.sync_copy(zt, cntsh.at[pl.ds(r0 + i * 16, 16)])
            return carry

        lax.fori_loop(0, RPT // 16, zb, 0)

        @pl.when(s == 0)
        def _():
            pltpu.sync_copy(zt, cntsh.at[pl.ds(NS * RPT, TAIL)])

        plsc.subcore_barrier()

        cbuf = (c0, c1)
        csem = (si0, si1)

        def fire(j, p):
            pltpu.make_async_copy(
                col_hbm.at[pl.ds(base + j * CH, CH)], cbuf[p], csem[p]).start()

        def drain_scat(j, p):
            pltpu.make_async_copy(
                col_hbm.at[pl.ds(base + j * CH, CH)], cbuf[p], csem[p]).wait()
            pltpu.sync_copy(ones, cntsh.at[cbuf[p]], add=True)

        fire(0, 0)

        def body(i, carry):
            j = 2 * i
            fire(j + 1, 1)
            drain_scat(j, 0)
            fire(j + 2, 0)
            drain_scat(j + 1, 1)
            return carry

        lax.fori_loop(0, (NCH - 1) // 2, body, 0)
        drain_scat(NCH - 1, 0)

        plsc.subcore_barrier()

        @pl.when(c == 0)
        def _():
            pltpu.sync_copy(cntsh.at[pl.ds(r0, RPT)],
                            ca_hbm.at[pl.ds(r0, RPT)])

            @pl.when(s == 0)
            def _():
                pltpu.sync_copy(cntsh.at[pl.ds(NS * RPT, TAIL)],
                                ca_hbm.at[pl.ds(NS * RPT, TAIL)])

        @pl.when(c == 1)
        def _():
            pltpu.sync_copy(cntsh.at[pl.ds(r0, RPT)],
                            cb_hbm.at[pl.ds(r0, RPT)])

            @pl.when(s == 0)
            def _():
                pltpu.sync_copy(cntsh.at[pl.ds(NS * RPT, TAIL)],
                                cb_hbm.at[pl.ds(NS * RPT, TAIL)])

    return k(col)


def kernel(x, edge_index, edge_attr, ew1, eb1, ew2, eb2,
           n1w1, n1b1, n1w2, n1b2, n2w1, n2b1, n2w2, n2b2):
    row = edge_index[0]
    col = edge_index[1]

    wrow = jnp.concatenate([ew1[:D], n1w1[:D]], axis=1)
    brow = jnp.concatenate([eb1, n1b1])[None, :]
    wcol = ew1[D:2 * D]

    bf = jnp.bfloat16
    cnta, cntb = _sc_count(col)
    tabrow, tabcol = _pre(x, wrow, brow, wcol)
    gsrc, gdst = _sc_gather(tabrow, tabcol, row, col)
    e_new, msg = _edge(gsrc, gdst, edge_attr,
                       ew1[2 * D:].astype(bf), ew2.astype(bf), eb2[None, :],
                       n1w1[D:].astype(bf), n1w2.astype(bf), n1b2[None, :])
    agg = _sc_scatter(msg, col)
    x_new = _node(x, agg, cnta, cntb, n2w1[:D], n2w1[D:], n2b1[None, :],
                  n2w2, n2b2[None, :])
    return (x_new, e_new)


# speedup vs baseline: 3.9030x; 1.0039x over previous
import functools

import jax
import jax.numpy as jnp
from jax import lax
from jax.experimental import pallas as pl
from jax.experimental.pallas import tpu as pltpu
from jax.experimental.pallas import tpu_sc as plsc

N = 10000
E = 160000
D = 256

NC = 2
NS = 16
NW = NC * NS
EPW = E // NW
CH = 40
NCH = EPW // CH
RPT = 624
TAIL = N - NS * RPT
DC = D // NC

_BN = 2000
_BE = 2000

EPT = E // NS
NCHT = EPT // CH


def _pack2(lo, hi):
    lo16 = jax.lax.bitcast_convert_type(lo.astype(jnp.bfloat16), jnp.uint16)
    hi16 = jax.lax.bitcast_convert_type(hi.astype(jnp.bfloat16), jnp.uint16)
    return lo16.astype(jnp.int32) | (hi16.astype(jnp.int32) << 16)


def _unpack2(w):
    wu = jax.lax.bitcast_convert_type(w, jnp.uint32)
    lo = jax.lax.bitcast_convert_type(
        (wu & 0xFFFF).astype(jnp.uint16), jnp.bfloat16)
    hi = jax.lax.bitcast_convert_type(
        (wu >> 16).astype(jnp.uint16), jnp.bfloat16)
    return lo.astype(jnp.float32), hi.astype(jnp.float32)


def _pre_body(x_ref, wr_ref, br_ref, wc_ref, tr_ref, tc_ref):
    x = x_ref[...]
    tr = x @ wr_ref[...] + br_ref[...]
    tr_ref[...] = _pack2(tr[:, :D], tr[:, D:])
    tc = x @ wc_ref[...]
    tc_ref[...] = _pack2(tc[:, :DC], tc[:, DC:])


def _pre(x, wrow, brow, wcol):
    return pl.pallas_call(
        _pre_body,
        grid=(N // _BN,),
        in_specs=[
            pl.BlockSpec((_BN, D), lambda i: (i, 0)),
            pl.BlockSpec((D, 2 * D), lambda i: (0, 0)),
            pl.BlockSpec((1, 2 * D), lambda i: (0, 0)),
            pl.BlockSpec((D, D), lambda i: (0, 0)),
        ],
        out_specs=[
            pl.BlockSpec((_BN, D), lambda i: (i, 0)),
            pl.BlockSpec((_BN, DC), lambda i: (i, 0)),
        ],
        out_shape=[
            jax.ShapeDtypeStruct((N, D), jnp.int32),
            jax.ShapeDtypeStruct((N, DC), jnp.int32),
        ],
    )(x, wrow, brow, wcol)


def _edge_body(gs_ref, gd_ref, ea_ref, wea_ref, ew2_ref, eb2_ref,
               wen_ref, n1w2_ref, n1b2_ref, en_ref, msg_ref):
    a, m = _unpack2(gs_ref[...])
    gdlo, gdhi = _unpack2(gd_ref[...])
    gd = jnp.concatenate([gdlo, gdhi], axis=1)
    f32 = jnp.float32
    bf = jnp.bfloat16
    h1 = jnp.maximum(
        a + gd + jnp.dot(ea_ref[...].astype(bf), wea_ref[...],
                         preferred_element_type=f32),
        0.0)
    en = jnp.dot(h1.astype(bf), ew2_ref[...], preferred_element_type=f32)
    en = en + eb2_ref[...]
    en_ref[...] = en
    h2 = jnp.maximum(
        m + jnp.dot(en.astype(bf), wen_ref[...], preferred_element_type=f32),
        0.0)
    msg = jnp.dot(
        h2.astype(bf), n1w2_ref[...], preferred_element_type=f32) + n1b2_ref[...]
    msg_ref[0] = msg[:, :DC]
    msg_ref[1] = msg[:, DC:]


def _edge(gsrc, gdst, ea, wea, ew2, eb2, wen, n1w2, n1b2):
    wspec = pl.BlockSpec((D, D), lambda i: (0, 0))
    bspec = pl.BlockSpec((1, D), lambda i: (0, 0))
    return pl.pallas_call(
        _edge_body,
        grid=(E // _BE,),
        in_specs=[
            pl.BlockSpec((_BE, D), lambda i: (i, 0)),
            pl.BlockSpec((_BE, DC), lambda i: (i, 0)),
            pl.BlockSpec((_BE, D), lambda i: (i, 0)),
            wspec, wspec, bspec, wspec, wspec, bspec,
        ],
        out_specs=[
            pl.BlockSpec((_BE, D), lambda i: (i, 0)),
            pl.BlockSpec((2, _BE, DC), lambda i: (0, i, 0)),
        ],
        out_shape=[
            jax.ShapeDtypeStruct((E, D), jnp.float32),
            jax.ShapeDtypeStruct((2, E, DC), jnp.float32),
        ],
    )(gsrc, gdst, ea, wea, ew2, eb2, wen, n1w2, n1b2)


def _node_body(x_ref, agg_ref, ca_ref, cb_ref, w2a_ref, w2b_ref, b1_ref,
               w2_ref, b2_ref, out_ref):
    cnt = ca_ref[:, :1] + cb_ref[:, :1]
    agg = jnp.concatenate([agg_ref[0], agg_ref[1]], axis=1)
    mean = agg / jnp.maximum(cnt, 1.0)
    h = jnp.maximum(
        x_ref[...] @ w2a_ref[...] + mean @ w2b_ref[...] + b1_ref[...], 0.0)
    out_ref[...] = h @ w2_ref[...] + b2_ref[...]


def _node(x, agg, cnta, cntb, w2a, w2b, b1, w2, b2):
    wspec = pl.BlockSpec((D, D), lambda i: (0, 0))
    bspec = pl.BlockSpec((1, D), lambda i: (0, 0))
    return pl.pallas_call(
        _node_body,
        grid=(N // _BN,),
        in_specs=[
            pl.BlockSpec((_BN, D), lambda i: (i, 0)),
            pl.BlockSpec((2, _BN, DC), lambda i: (0, i, 0)),
            pl.BlockSpec((_BN, DC), lambda i: (i, 0)),
            pl.BlockSpec((_BN, DC), lambda i: (i, 0)),
            wspec, wspec, bspec, wspec, bspec,
        ],
        out_specs=pl.BlockSpec((_BN, D), lambda i: (i, 0)),
        out_shape=jax.ShapeDtypeStruct((N, D), jnp.float32),
    )(x, agg, cnta, cntb, w2a, w2b, b1, w2, b2)


def _sc_gather(tabrow, tabcol, row, col):
    mesh = plsc.VectorSubcoreMesh(core_axis_name="c", subcore_axis_name="s")

    @functools.partial(
        pl.kernel,
        out_type=[
            jax.ShapeDtypeStruct((E, D), jnp.int32),
            jax.ShapeDtypeStruct((E, DC), jnp.int32),
        ],
        mesh=mesh,
        scratch_types=[
            pltpu.VMEM((EPW,), jnp.int32),
            pltpu.VMEM((EPW,), jnp.int32),
            pltpu.VMEM((CH, D), jnp.int32),
            pltpu.VMEM((CH, D), jnp.int32),
            pltpu.VMEM((CH, DC), jnp.int32),
            pltpu.VMEM((CH, DC), jnp.int32),
            pltpu.SemaphoreType.DMA,
            pltpu.SemaphoreType.DMA,
            pltpu.SemaphoreType.DMA,
            pltpu.SemaphoreType.DMA,
        ],
    )
    def k(tr_hbm, tc_hbm, row_hbm, col_hbm, gs_hbm, gd_hbm,
          rowv, colv, a0, a1, b0, b1, sa0, sa1, sb0, sb1):
        wid = lax.axis_index("s") * NC + lax.axis_index("c")
        base = wid * EPW
        pltpu.sync_copy(row_hbm.at[pl.ds(base, EPW)], rowv)
        pltpu.sync_copy(col_hbm.at[pl.ds(base, EPW)], colv)

        abuf = (a0, a1)
        bbuf = (b0, b1)
        asem = (sa0, sa1)
        bsem = (sb0, sb1)

        def fire(j, p):
            off = j * CH
            pltpu.make_async_copy(
                tr_hbm.at[rowv.at[pl.ds(off, CH)]], abuf[p], asem[p]).start()
            pltpu.make_async_copy(
                tc_hbm.at[colv.at[pl.ds(off, CH)]], bbuf[p], bsem[p]).start()

        def drain_out(j, p):
            off = j * CH
            pltpu.make_async_copy(
                tr_hbm.at[rowv.at[pl.ds(off, CH)]], abuf[p], asem[p]).wait()
            pltpu.sync_copy(abuf[p], gs_hbm.at[pl.ds(base + off, CH)])
            pltpu.make_async_copy(
                tc_hbm.at[colv.at[pl.ds(off, CH)]], bbuf[p], bsem[p]).wait()
            pltpu.sync_copy(bbuf[p], gd_hbm.at[pl.ds(base + off, CH)])

        fire(0, 0)

        def body(i, carry):
            j = 2 * i
            fire(j + 1, 1)
            drain_out(j, 0)
            fire(j + 2, 0)
            drain_out(j + 1, 1)
            return carry

        lax.fori_loop(0, (NCH - 1) // 2, body, 0)
        drain_out(NCH - 1, 0)

    return k(tabrow, tabcol, row, col)


def _sc_scatter(msg, col):
    mesh = plsc.VectorSubcoreMesh(core_axis_name="c", subcore_axis_name="s")

    @functools.partial(
        pl.kernel,
        out_type=jax.ShapeDtypeStruct((2, N, DC), jnp.float32),
        mesh=mesh,
        scratch_types=[
            pltpu.VMEM((CH,), jnp.int32),
            pltpu.VMEM((CH,), jnp.int32),
            pltpu.VMEM((CH, DC), jnp.float32),
            pltpu.VMEM((CH, DC), jnp.float32),
            pltpu.VMEM((16, DC), jnp.float32),
            pltpu.VMEM_SHARED((N, DC), jnp.float32),
            pltpu.SemaphoreType.DMA,
            pltpu.SemaphoreType.DMA,
            pltpu.SemaphoreType.DMA,
            pltpu.SemaphoreType.DMA,
        ],
    )
    def k(msg_hbm, col_hbm, agg_hbm,
          c0, c1, m0, m1, zt, aggsh, s0, s1, si0, si1):
        c = lax.axis_index("c")
        s = lax.axis_index("s")
        base = s * EPT
        r0 = s * RPT

        for r in range(16):
            for q in range(DC // 16):
                zt[r, pl.ds(q * 16, 16)] = jnp.zeros((16,), jnp.float32)

        def zb(i, carry):
            pltpu.sync_copy(zt, aggsh.at[pl.ds(r0 + i * 16, 16)])
            return carry

        lax.fori_loop(0, RPT // 16, zb, 0)

        @pl.when(s == 0)
        def _():
            pltpu.sync_copy(zt, aggsh.at[pl.ds(NS * RPT, TAIL)])

        plsc.subcore_barrier()

        mbuf = (m0, m1)
        msem = (s0, s1)
        cbuf = (c0, c1)
        csem = (si0, si1)

        def fire(j, p):
            pltpu.make_async_copy(
                col_hbm.at[pl.ds(base + j * CH, CH)], cbuf[p], csem[p]).start()
            pltpu.make_async_copy(
                msg_hbm.at[c, pl.ds(base + j * CH, CH)],
                mbuf[p], msem[p]).start()

        def drain_scat(j, p):
            pltpu.make_async_copy(
                col_hbm.at[pl.ds(base + j * CH, CH)], cbuf[p], csem[p]).wait()
            pltpu.make_async_copy(
                msg_hbm.at[c, pl.ds(base + j * CH, CH)],
                mbuf[p], msem[p]).wait()
            pltpu.sync_copy(mbuf[p], aggsh.at[cbuf[p]], add=True)

        fire(0, 0)

        def body(i, carry):
            j = 2 * i
            fire(j + 1, 1)
            drain_scat(j, 0)
            fire(j + 2, 0)
            drain_scat(j + 1, 1)
            return carry

        lax.fori_loop(0, (NCHT - 2) // 2, body, 0)
        fire(NCHT - 1, 1)
        drain_scat(NCHT - 2, 0)
        drain_scat(NCHT - 1, 1)

        plsc.subcore_barrier()

        pltpu.sync_copy(aggsh.at[pl.ds(r0, RPT)],
                        agg_hbm.at[c, pl.ds(r0, RPT)])

        @pl.when(s == 0)
        def _():
            pltpu.sync_copy(aggsh.at[pl.ds(NS * RPT, TAIL)],
                            agg_hbm.at[c, pl.ds(NS * RPT, TAIL)])

    return k(msg, col)


def _sc_count(col):
    mesh = plsc.VectorSubcoreMesh(core_axis_name="c", subcore_axis_name="s")

    @functools.partial(
        pl.kernel,
        out_type=[
            jax.ShapeDtypeStruct((N, DC), jnp.float32),
            jax.ShapeDtypeStruct((N, DC), jnp.float32),
        ],
        mesh=mesh,
        scratch_types=[
            pltpu.VMEM((CH,), jnp.int32),
            pltpu.VMEM((CH,), jnp.int32),
            pltpu.VMEM((CH, DC), jnp.float32),
            pltpu.VMEM((16, DC), jnp.float32),
            pltpu.VMEM_SHARED((N, DC), jnp.float32),
            pltpu.SemaphoreType.DMA,
            pltpu.SemaphoreType.DMA,
        ],
    )
    def k(col_hbm, ca_hbm, cb_hbm, c0, c1, ones, zt, cntsh, si0, si1):
        c = lax.axis_index("c")
        s = lax.axis_index("s")
        wid = s * NC + c
        base = wid * EPW
        r0 = s * RPT

        for r in range(16):
            for q in range(DC // 16):
                zt[r, pl.ds(q * 16, 16)] = jnp.zeros((16,), jnp.float32)
        for r in range(CH):
            for q in range(DC // 16):
                ones[r, pl.ds(q * 16, 16)] = jnp.ones((16,), jnp.float32)

        def zb(i, carry):
            pltpu.sync_copy(zt, cntsh.at[pl.ds(r0 + i * 16, 16)])
            return carry

        lax.fori_loop(0, RPT // 16, zb, 0)

        @pl.when(s == 0)
        def _():
            pltpu.sync_copy(zt, cntsh.at[pl.ds(NS * RPT, TAIL)])

        plsc.subcore_barrier()

        cbuf = (c0, c1)
        csem = (si0, si1)

        def fire(j, p):
            pltpu.make_async_copy(
                col_hbm.at[pl.ds(base + j * CH, CH)], cbuf[p], csem[p]).start()

        def drain_scat(j, p):
            pltpu.make_async_copy(
                col_hbm.at[pl.ds(base + j * CH, CH)], cbuf[p], csem[p]).wait()
            pltpu.sync_copy(ones, cntsh.at[cbuf[p]], add=True)

        fire(0, 0)

        def body(i, carry):
            j = 2 * i
            fire(j + 1, 1)
            drain_scat(j, 0)
            fire(j + 2, 0)
            drain_scat(j + 1, 1)
            return carry

        lax.fori_loop(0, (NCH - 1) // 2, body, 0)
        drain_scat(NCH - 1, 0)

        plsc.subcore_barrier()

        @pl.when(c == 0)
        def _():
            pltpu.sync_copy(cntsh.at[pl.ds(r0, RPT)],
                            ca_hbm.at[pl.ds(r0, RPT)])

            @pl.when(s == 0)
            def _():
                pltpu.sync_copy(cntsh.at[pl.ds(NS * RPT, TAIL)],
                                ca_hbm.at[pl.ds(NS * RPT, TAIL)])

        @pl.when(c == 1)
        def _():
            pltpu.sync_copy(cntsh.at[pl.ds(r0, RPT)],
                            cb_hbm.at[pl.ds(r0, RPT)])

            @pl.when(s == 0)
            def _():
                pltpu.sync_copy(cntsh.at[pl.ds(NS * RPT, TAIL)],
                                cb_hbm.at[pl.ds(NS * RPT, TAIL)])

    return k(col)


def kernel(x, edge_index, edge_attr, ew1, eb1, ew2, eb2,
           n1w1, n1b1, n1w2, n1b2, n2w1, n2b1, n2w2, n2b2):
    row = edge_index[0]
    col = edge_index[1]

    wrow = jnp.concatenate([ew1[:D], n1w1[:D]], axis=1)
    brow = jnp.concatenate([eb1, n1b1])[None, :]
    wcol = ew1[D:2 * D]

    bf = jnp.bfloat16
    cnta, cntb = _sc_count(col)
    tabrow, tabcol = _pre(x, wrow, brow, wcol)
    gsrc, gdst = _sc_gather(tabrow, tabcol, row, col)
    e_new, msg = _edge(gsrc, gdst, edge_attr,
                       ew1[2 * D:].astype(bf), ew2.astype(bf), eb2[None, :],
                       n1w1[D:].astype(bf), n1w2.astype(bf), n1b2[None, :])
    agg = _sc_scatter(msg, col)
    x_new = _node(x, agg, cnta, cntb, n2w1[:D], n2w1[D:], n2b1[None, :],
                  n2w2, n2b2[None, :])
    return (x_new, e_new)


# speedup vs baseline: 4.1686x; 1.0680x over previous
import functools

import jax
import jax.numpy as jnp
from jax import lax
from jax.experimental import pallas as pl
from jax.experimental.pallas import tpu as pltpu
from jax.experimental.pallas import tpu_sc as plsc

N = 10000
E = 160000
D = 256

NC = 2
NS = 16
NW = NC * NS
EPW = E // NW
CH = 40
NCH = EPW // CH
RPT = 624
TAIL = N - NS * RPT
DC = D // NC

_BN = 2000
_BE = 2000

EPT = E // NS
CHS = 80
NCHT = EPT // CHS


def _pack2(lo, hi):
    lo16 = jax.lax.bitcast_convert_type(lo.astype(jnp.bfloat16), jnp.uint16)
    hi16 = jax.lax.bitcast_convert_type(hi.astype(jnp.bfloat16), jnp.uint16)
    return lo16.astype(jnp.int32) | (hi16.astype(jnp.int32) << 16)


def _unpack2(w):
    wu = jax.lax.bitcast_convert_type(w, jnp.uint32)
    lo = jax.lax.bitcast_convert_type(
        (wu & 0xFFFF).astype(jnp.uint16), jnp.bfloat16)
    hi = jax.lax.bitcast_convert_type(
        (wu >> 16).astype(jnp.uint16), jnp.bfloat16)
    return lo.astype(jnp.float32), hi.astype(jnp.float32)


def _pre_body(x_ref, wr_ref, br_ref, wc_ref, tr_ref, tc_ref):
    x = x_ref[...]
    tr = x @ wr_ref[...] + br_ref[...]
    tr_ref[...] = _pack2(tr[:, :D], tr[:, D:])
    tc = x @ wc_ref[...]
    tc_ref[...] = _pack2(tc[:, :DC], tc[:, DC:])


def _pre(x, wrow, brow, wcol):
    return pl.pallas_call(
        _pre_body,
        grid=(N // _BN,),
        in_specs=[
            pl.BlockSpec((_BN, D), lambda i: (i, 0)),
            pl.BlockSpec((D, 2 * D), lambda i: (0, 0)),
            pl.BlockSpec((1, 2 * D), lambda i: (0, 0)),
            pl.BlockSpec((D, D), lambda i: (0, 0)),
        ],
        out_specs=[
            pl.BlockSpec((_BN, D), lambda i: (i, 0)),
            pl.BlockSpec((_BN, DC), lambda i: (i, 0)),
        ],
        out_shape=[
            jax.ShapeDtypeStruct((N, D), jnp.int32),
            jax.ShapeDtypeStruct((N, DC), jnp.int32),
        ],
    )(x, wrow, brow, wcol)


def _edge_body(gs_ref, gd_ref, ea_ref, wea_ref, ew2_ref, eb2_ref,
               wen_ref, n1w2_ref, n1b2_ref, en_ref, msg_ref):
    a, m = _unpack2(gs_ref[...])
    gdlo, gdhi = _unpack2(gd_ref[...])
    gd = jnp.concatenate([gdlo, gdhi], axis=1)
    f32 = jnp.float32
    bf = jnp.bfloat16
    h1 = jnp.maximum(
        a + gd + jnp.dot(ea_ref[...].astype(bf), wea_ref[...],
                         preferred_element_type=f32),
        0.0)
    en = jnp.dot(h1.astype(bf), ew2_ref[...], preferred_element_type=f32)
    en = en + eb2_ref[...]
    en_ref[...] = en
    h2 = jnp.maximum(
        m + jnp.dot(en.astype(bf), wen_ref[...], preferred_element_type=f32),
        0.0)
    msg = jnp.dot(
        h2.astype(bf), n1w2_ref[...], preferred_element_type=f32) + n1b2_ref[...]
    msg_ref[0] = msg[:, :DC]
    msg_ref[1] = msg[:, DC:]


def _edge(gsrc, gdst, ea, wea, ew2, eb2, wen, n1w2, n1b2):
    wspec = pl.BlockSpec((D, D), lambda i: (0, 0))
    bspec = pl.BlockSpec((1, D), lambda i: (0, 0))
    return pl.pallas_call(
        _edge_body,
        grid=(E // _BE,),
        in_specs=[
            pl.BlockSpec((_BE, D), lambda i: (i, 0)),
            pl.BlockSpec((_BE, DC), lambda i: (i, 0)),
            pl.BlockSpec((_BE, D), lambda i: (i, 0)),
            wspec, wspec, bspec, wspec, wspec, bspec,
        ],
        out_specs=[
            pl.BlockSpec((_BE, D), lambda i: (i, 0)),
            pl.BlockSpec((2, _BE, DC), lambda i: (0, i, 0)),
        ],
        out_shape=[
            jax.ShapeDtypeStruct((E, D), jnp.float32),
            jax.ShapeDtypeStruct((2, E, DC), jnp.float32),
        ],
    )(gsrc, gdst, ea, wea, ew2, eb2, wen, n1w2, n1b2)


def _node_body(x_ref, agg_ref, ca_ref, cb_ref, w2a_ref, w2b_ref, b1_ref,
               w2_ref, b2_ref, out_ref):
    cnt = ca_ref[:, :1] + cb_ref[:, :1]
    agg = jnp.concatenate([agg_ref[0], agg_ref[1]], axis=1)
    mean = agg / jnp.maximum(cnt, 1.0)
    h = jnp.maximum(
        x_ref[...] @ w2a_ref[...] + mean @ w2b_ref[...] + b1_ref[...], 0.0)
    out_ref[...] = h @ w2_ref[...] + b2_ref[...]


def _node(x, agg, cnta, cntb, w2a, w2b, b1, w2, b2):
    wspec = pl.BlockSpec((D, D), lambda i: (0, 0))
    bspec = pl.BlockSpec((1, D), lambda i: (0, 0))
    return pl.pallas_call(
        _node_body,
        grid=(N // _BN,),
        in_specs=[
            pl.BlockSpec((_BN, D), lambda i: (i, 0)),
            pl.BlockSpec((2, _BN, DC), lambda i: (0, i, 0)),
            pl.BlockSpec((_BN, DC), lambda i: (i, 0)),
            pl.BlockSpec((_BN, DC), lambda i: (i, 0)),
            wspec, wspec, bspec, wspec, bspec,
        ],
        out_specs=pl.BlockSpec((_BN, D), lambda i: (i, 0)),
        out_shape=jax.ShapeDtypeStruct((N, D), jnp.float32),
    )(x, agg, cnta, cntb, w2a, w2b, b1, w2, b2)


def _sc_gather(tabrow, tabcol, row, col):
    mesh = plsc.VectorSubcoreMesh(core_axis_name="c", subcore_axis_name="s")

    @functools.partial(
        pl.kernel,
        out_type=[
            jax.ShapeDtypeStruct((E, D), jnp.int32),
            jax.ShapeDtypeStruct((E, DC), jnp.int32),
        ],
        mesh=mesh,
        scratch_types=[
            pltpu.VMEM((EPW,), jnp.int32),
            pltpu.VMEM((EPW,), jnp.int32),
            pltpu.VMEM((CH, D), jnp.int32),
            pltpu.VMEM((CH, D), jnp.int32),
            pltpu.VMEM((CH, DC), jnp.int32),
            pltpu.VMEM((CH, DC), jnp.int32),
            pltpu.SemaphoreType.DMA,
            pltpu.SemaphoreType.DMA,
            pltpu.SemaphoreType.DMA,
            pltpu.SemaphoreType.DMA,
        ],
    )
    def k(tr_hbm, tc_hbm, row_hbm, col_hbm, gs_hbm, gd_hbm,
          rowv, colv, a0, a1, b0, b1, sa0, sa1, sb0, sb1):
        wid = lax.axis_index("s") * NC + lax.axis_index("c")
        base = wid * EPW
        pltpu.sync_copy(row_hbm.at[pl.ds(base, EPW)], rowv)
        pltpu.sync_copy(col_hbm.at[pl.ds(base, EPW)], colv)

        abuf = (a0, a1)
        bbuf = (b0, b1)
        asem = (sa0, sa1)
        bsem = (sb0, sb1)

        def fire(j, p):
            off = j * CH
            pltpu.make_async_copy(
                tr_hbm.at[rowv.at[pl.ds(off, CH)]], abuf[p], asem[p]).start()
            pltpu.make_async_copy(
                tc_hbm.at[colv.at[pl.ds(off, CH)]], bbuf[p], bsem[p]).start()

        def drain_out(j, p):
            off = j * CH
            pltpu.make_async_copy(
                tr_hbm.at[rowv.at[pl.ds(off, CH)]], abuf[p], asem[p]).wait()
            pltpu.sync_copy(abuf[p], gs_hbm.at[pl.ds(base + off, CH)])
            pltpu.make_async_copy(
                tc_hbm.at[colv.at[pl.ds(off, CH)]], bbuf[p], bsem[p]).wait()
            pltpu.sync_copy(bbuf[p], gd_hbm.at[pl.ds(base + off, CH)])

        fire(0, 0)

        def body(i, carry):
            j = 2 * i
            fire(j + 1, 1)
            drain_out(j, 0)
            fire(j + 2, 0)
            drain_out(j + 1, 1)
            return carry

        lax.fori_loop(0, (NCH - 1) // 2, body, 0)
        drain_out(NCH - 1, 0)

    return k(tabrow, tabcol, row, col)


def _sc_scatter(msg, col):
    mesh = plsc.VectorSubcoreMesh(core_axis_name="c", subcore_axis_name="s")

    @functools.partial(
        pl.kernel,
        out_type=jax.ShapeDtypeStruct((2, N, DC), jnp.float32),
        mesh=mesh,
        scratch_types=[
            pltpu.VMEM((CHS,), jnp.int32),
            pltpu.VMEM((CHS,), jnp.int32),
            pltpu.VMEM((CHS, DC), jnp.float32),
            pltpu.VMEM((CHS, DC), jnp.float32),
            pltpu.VMEM((16, DC), jnp.float32),
            pltpu.VMEM_SHARED((N, DC), jnp.float32),
            pltpu.SemaphoreType.DMA,
            pltpu.SemaphoreType.DMA,
            pltpu.SemaphoreType.DMA,
            pltpu.SemaphoreType.DMA,
        ],
    )
    def k(msg_hbm, col_hbm, agg_hbm,
          c0, c1, m0, m1, zt, aggsh, s0, s1, si0, si1):
        c = lax.axis_index("c")
        s = lax.axis_index("s")
        base = s * EPT
        r0 = s * RPT

        for r in range(16):
            for q in range(DC // 16):
                zt[r, pl.ds(q * 16, 16)] = jnp.zeros((16,), jnp.float32)

        def zb(i, carry):
            pltpu.sync_copy(zt, aggsh.at[pl.ds(r0 + i * 16, 16)])
            return carry

        lax.fori_loop(0, RPT // 16, zb, 0)

        @pl.when(s == 0)
        def _():
            pltpu.sync_copy(zt, aggsh.at[pl.ds(NS * RPT, TAIL)])

        plsc.subcore_barrier()

        mbuf = (m0, m1)
        msem = (s0, s1)
        cbuf = (c0, c1)
        csem = (si0, si1)

        def fire(j, p):
            pltpu.make_async_copy(
                col_hbm.at[pl.ds(base + j * CHS, CHS)], cbuf[p], csem[p]).start()
            pltpu.make_async_copy(
                msg_hbm.at[c, pl.ds(base + j * CHS, CHS)],
                mbuf[p], msem[p]).start()

        def drain_scat(j, p):
            pltpu.make_async_copy(
                col_hbm.at[pl.ds(base + j * CHS, CHS)], cbuf[p], csem[p]).wait()
            pltpu.make_async_copy(
                msg_hbm.at[c, pl.ds(base + j * CHS, CHS)],
                mbuf[p], msem[p]).wait()
            pltpu.sync_copy(mbuf[p], aggsh.at[cbuf[p]], add=True)

        fire(0, 0)

        def body(i, carry):
            j = 2 * i
            fire(j + 1, 1)
            drain_scat(j, 0)
            fire(j + 2, 0)
            drain_scat(j + 1, 1)
            return carry

        lax.fori_loop(0, (NCHT - 1) // 2, body, 0)
        drain_scat(NCHT - 1, 0)

        plsc.subcore_barrier()

        pltpu.sync_copy(aggsh.at[pl.ds(r0, RPT)],
                        agg_hbm.at[c, pl.ds(r0, RPT)])

        @pl.when(s == 0)
        def _():
            pltpu.sync_copy(aggsh.at[pl.ds(NS * RPT, TAIL)],
                            agg_hbm.at[c, pl.ds(NS * RPT, TAIL)])

    return k(msg, col)


def _sc_count(col):
    mesh = plsc.VectorSubcoreMesh(core_axis_name="c", subcore_axis_name="s")

    @functools.partial(
        pl.kernel,
        out_type=[
            jax.ShapeDtypeStruct((N, DC), jnp.float32),
            jax.ShapeDtypeStruct((N, DC), jnp.float32),
        ],
        mesh=mesh,
        scratch_types=[
            pltpu.VMEM((CH,), jnp.int32),
            pltpu.VMEM((CH,), jnp.int32),
            pltpu.VMEM((CH, DC), jnp.float32),
            pltpu.VMEM((16, DC), jnp.float32),
            pltpu.VMEM_SHARED((N, DC), jnp.float32),
            pltpu.SemaphoreType.DMA,
            pltpu.SemaphoreType.DMA,
        ],
    )
    def k(col_hbm, ca_hbm, cb_hbm, c0, c1, ones, zt, cntsh, si0, si1):
        c = lax.axis_index("c")
        s = lax.axis_index("s")
        wid = s * NC + c
        base = wid * EPW
        r0 = s * RPT

        for r in range(16):
            for q in range(DC // 16):
                zt[r, pl.ds(q * 16, 16)] = jnp.zeros((16,), jnp.float32)
        for r in range(CH):
            for q in range(DC // 16):
                ones[r, pl.ds(q * 16, 16)] = jnp.ones((16,), jnp.float32)

        def zb(i, carry):
            pltpu.sync_copy(zt, cntsh.at[pl.ds(r0 + i * 16, 16)])
            return carry

        lax.fori_loop(0, RPT // 16, zb, 0)

        @pl.when(s == 0)
        def _():
            pltpu.sync_copy(zt, cntsh.at[pl.ds(NS * RPT, TAIL)])

        plsc.subcore_barrier()

        cbuf = (c0, c1)
        csem = (si0, si1)

        def fire(j, p):
            pltpu.make_async_copy(
                col_hbm.at[pl.ds(base + j * CH, CH)], cbuf[p], csem[p]).start()

        def drain_scat(j, p):
            pltpu.make_async_copy(
                col_hbm.at[pl.ds(base + j * CH, CH)], cbuf[p], csem[p]).wait()
            pltpu.sync_copy(ones, cntsh.at[cbuf[p]], add=True)

        fire(0, 0)

        def body(i, carry):
            j = 2 * i
            fire(j + 1, 1)
            drain_scat(j, 0)
            fire(j + 2, 0)
            drain_scat(j + 1, 1)
            return carry

        lax.fori_loop(0, (NCH - 1) // 2, body, 0)
        drain_scat(NCH - 1, 0)

        plsc.subcore_barrier()

        @pl.when(c == 0)
        def _():
            pltpu.sync_copy(cntsh.at[pl.ds(r0, RPT)],
                            ca_hbm.at[pl.ds(r0, RPT)])

            @pl.when(s == 0)
            def _():
                pltpu.sync_copy(cntsh.at[pl.ds(NS * RPT, TAIL)],
                                ca_hbm.at[pl.ds(NS * RPT, TAIL)])

        @pl.when(c == 1)
        def _():
            pltpu.sync_copy(cntsh.at[pl.ds(r0, RPT)],
                            cb_hbm.at[pl.ds(r0, RPT)])

            @pl.when(s == 0)
            def _():
                pltpu.sync_copy(cntsh.at[pl.ds(NS * RPT, TAIL)],
                                cb_hbm.at[pl.ds(NS * RPT, TAIL)])

    return k(col)


def kernel(x, edge_index, edge_attr, ew1, eb1, ew2, eb2,
           n1w1, n1b1, n1w2, n1b2, n2w1, n2b1, n2w2, n2b2):
    row = edge_index[0]
    col = edge_index[1]

    wrow = jnp.concatenate([ew1[:D], n1w1[:D]], axis=1)
    brow = jnp.concatenate([eb1, n1b1])[None, :]
    wcol = ew1[D:2 * D]

    bf = jnp.bfloat16
    cnta, cntb = _sc_count(col)
    tabrow, tabcol = _pre(x, wrow, brow, wcol)
    gsrc, gdst = _sc_gather(tabrow, tabcol, row, col)
    e_new, msg = _edge(gsrc, gdst, edge_attr,
                       ew1[2 * D:].astype(bf), ew2.astype(bf), eb2[None, :],
                       n1w1[D:].astype(bf), n1w2.astype(bf), n1b2[None, :])
    agg = _sc_scatter(msg, col)
    x_new = _node(x, agg, cnta, cntb, n2w1[:D], n2w1[D:], n2b1[None, :],
                  n2w2, n2b2[None, :])
    return (x_new, e_new)


# speedup vs baseline: 4.1731x; 1.0011x over previous
import functools

import jax
import jax.numpy as jnp
from jax import lax
from jax.experimental import pallas as pl
from jax.experimental.pallas import tpu as pltpu
from jax.experimental.pallas import tpu_sc as plsc

N = 10000
E = 160000
D = 256

NC = 2
NS = 16
NW = NC * NS
EPW = E // NW
CH = 40
NCH = EPW // CH
RPT = 624
TAIL = N - NS * RPT
DC = D // NC

_BN = 2000
_BE = 2000

EPT = E // NS
CHS = 80
NCHT = EPT // CHS
CHG = 128
GT = EPW - (EPW // CHG) * CHG


def _pack2(lo, hi):
    lo16 = jax.lax.bitcast_convert_type(lo.astype(jnp.bfloat16), jnp.uint16)
    hi16 = jax.lax.bitcast_convert_type(hi.astype(jnp.bfloat16), jnp.uint16)
    return lo16.astype(jnp.int32) | (hi16.astype(jnp.int32) << 16)


def _unpack2(w):
    wu = jax.lax.bitcast_convert_type(w, jnp.uint32)
    lo = jax.lax.bitcast_convert_type(
        (wu & 0xFFFF).astype(jnp.uint16), jnp.bfloat16)
    hi = jax.lax.bitcast_convert_type(
        (wu >> 16).astype(jnp.uint16), jnp.bfloat16)
    return lo.astype(jnp.float32), hi.astype(jnp.float32)


def _pre_body(x_ref, wr_ref, br_ref, wc_ref, tr_ref, tc_ref):
    x = x_ref[...]
    tr = x @ wr_ref[...] + br_ref[...]
    tr_ref[...] = _pack2(tr[:, :D], tr[:, D:])
    tc = x @ wc_ref[...]
    tc_ref[...] = _pack2(tc[:, :DC], tc[:, DC:])


def _pre(x, wrow, brow, wcol):
    return pl.pallas_call(
        _pre_body,
        grid=(N // _BN,),
        in_specs=[
            pl.BlockSpec((_BN, D), lambda i: (i, 0)),
            pl.BlockSpec((D, 2 * D), lambda i: (0, 0)),
            pl.BlockSpec((1, 2 * D), lambda i: (0, 0)),
            pl.BlockSpec((D, D), lambda i: (0, 0)),
        ],
        out_specs=[
            pl.BlockSpec((_BN, D), lambda i: (i, 0)),
            pl.BlockSpec((_BN, DC), lambda i: (i, 0)),
        ],
        out_shape=[
            jax.ShapeDtypeStruct((N, D), jnp.int32),
            jax.ShapeDtypeStruct((N, DC), jnp.int32),
        ],
    )(x, wrow, brow, wcol)


def _edge_body(gs_ref, gd_ref, ea_ref, wea_ref, ew2_ref, eb2_ref,
               wen_ref, n1w2_ref, n1b2_ref, en_ref, msg_ref):
    a, m = _unpack2(gs_ref[...])
    gdlo, gdhi = _unpack2(gd_ref[...])
    gd = jnp.concatenate([gdlo, gdhi], axis=1)
    f32 = jnp.float32
    bf = jnp.bfloat16
    h1 = jnp.maximum(
        a + gd + jnp.dot(ea_ref[...].astype(bf), wea_ref[...],
                         preferred_element_type=f32),
        0.0)
    en = jnp.dot(h1.astype(bf), ew2_ref[...], preferred_element_type=f32)
    en = en + eb2_ref[...]
    en_ref[...] = en
    h2 = jnp.maximum(
        m + jnp.dot(en.astype(bf), wen_ref[...], preferred_element_type=f32),
        0.0)
    msg = jnp.dot(
        h2.astype(bf), n1w2_ref[...], preferred_element_type=f32) + n1b2_ref[...]
    msg_ref[0] = msg[:, :DC]
    msg_ref[1] = msg[:, DC:]


def _edge(gsrc, gdst, ea, wea, ew2, eb2, wen, n1w2, n1b2):
    wspec = pl.BlockSpec((D, D), lambda i: (0, 0))
    bspec = pl.BlockSpec((1, D), lambda i: (0, 0))
    return pl.pallas_call(
        _edge_body,
        grid=(E // _BE,),
        in_specs=[
            pl.BlockSpec((_BE, D), lambda i: (i, 0)),
            pl.BlockSpec((_BE, DC), lambda i: (i, 0)),
            pl.BlockSpec((_BE, D), lambda i: (i, 0)),
            wspec, wspec, bspec, wspec, wspec, bspec,
        ],
        out_specs=[
            pl.BlockSpec((_BE, D), lambda i: (i, 0)),
            pl.BlockSpec((2, _BE, DC), lambda i: (0, i, 0)),
        ],
        out_shape=[
            jax.ShapeDtypeStruct((E, D), jnp.float32),
            jax.ShapeDtypeStruct((2, E, DC), jnp.float32),
        ],
    )(gsrc, gdst, ea, wea, ew2, eb2, wen, n1w2, n1b2)


def _node_body(x_ref, agg_ref, ca_ref, cb_ref, w2a_ref, w2b_ref, b1_ref,
               w2_ref, b2_ref, out_ref):
    cnt = ca_ref[:, :1] + cb_ref[:, :1]
    agg = jnp.concatenate([agg_ref[0], agg_ref[1]], axis=1)
    mean = agg / jnp.maximum(cnt, 1.0)
    h = jnp.maximum(
        x_ref[...] @ w2a_ref[...] + mean @ w2b_ref[...] + b1_ref[...], 0.0)
    out_ref[...] = h @ w2_ref[...] + b2_ref[...]


def _node(x, agg, cnta, cntb, w2a, w2b, b1, w2, b2):
    wspec = pl.BlockSpec((D, D), lambda i: (0, 0))
    bspec = pl.BlockSpec((1, D), lambda i: (0, 0))
    return pl.pallas_call(
        _node_body,
        grid=(N // _BN,),
        in_specs=[
            pl.BlockSpec((_BN, D), lambda i: (i, 0)),
            pl.BlockSpec((2, _BN, DC), lambda i: (0, i, 0)),
            pl.BlockSpec((_BN, DC), lambda i: (i, 0)),
            pl.BlockSpec((_BN, DC), lambda i: (i, 0)),
            wspec, wspec, bspec, wspec, bspec,
        ],
        out_specs=pl.BlockSpec((_BN, D), lambda i: (i, 0)),
        out_shape=jax.ShapeDtypeStruct((N, D), jnp.float32),
    )(x, agg, cnta, cntb, w2a, w2b, b1, w2, b2)


def _sc_gather(tabrow, tabcol, row, col):
    mesh = plsc.VectorSubcoreMesh(core_axis_name="c", subcore_axis_name="s")

    @functools.partial(
        pl.kernel,
        out_type=[
            jax.ShapeDtypeStruct((E, D), jnp.int32),
            jax.ShapeDtypeStruct((E, DC), jnp.int32),
        ],
        mesh=mesh,
        scratch_types=[
            pltpu.VMEM((EPW,), jnp.int32),
            pltpu.VMEM((EPW,), jnp.int32),
            pltpu.VMEM((CHG, D), jnp.int32),
            pltpu.VMEM((CHG, D), jnp.int32),
            pltpu.VMEM((CHG, DC), jnp.int32),
            pltpu.VMEM((CHG, DC), jnp.int32),
            pltpu.VMEM((GT, D), jnp.int32),
            pltpu.VMEM((GT, DC), jnp.int32),
            pltpu.SemaphoreType.DMA,
            pltpu.SemaphoreType.DMA,
            pltpu.SemaphoreType.DMA,
            pltpu.SemaphoreType.DMA,
        ],
    )
    def k(tr_hbm, tc_hbm, row_hbm, col_hbm, gs_hbm, gd_hbm,
          rowv, colv, a0, a1, b0, b1, at_, bt_, sa0, sa1, sb0, sb1):
        wid = lax.axis_index("s") * NC + lax.axis_index("c")
        base = wid * EPW
        nbig = EPW // CHG
        pltpu.sync_copy(row_hbm.at[pl.ds(base, EPW)], rowv)
        pltpu.sync_copy(col_hbm.at[pl.ds(base, EPW)], colv)

        abuf = (a0, a1)
        bbuf = (b0, b1)
        asem = (sa0, sa1)
        bsem = (sb0, sb1)

        def fire(j, p):
            off = j * CHG
            pltpu.make_async_copy(
                tr_hbm.at[rowv.at[pl.ds(off, CHG)]], abuf[p], asem[p]).start()
            pltpu.make_async_copy(
                tc_hbm.at[colv.at[pl.ds(off, CHG)]], bbuf[p], bsem[p]).start()

        def drain_out(j, p):
            off = j * CHG
            pltpu.make_async_copy(
                tr_hbm.at[rowv.at[pl.ds(off, CHG)]], abuf[p], asem[p]).wait()
            pltpu.sync_copy(abuf[p], gs_hbm.at[pl.ds(base + off, CHG)])
            pltpu.make_async_copy(
                tc_hbm.at[colv.at[pl.ds(off, CHG)]], bbuf[p], bsem[p]).wait()
            pltpu.sync_copy(bbuf[p], gd_hbm.at[pl.ds(base + off, CHG)])

        fire(0, 0)

        def body(i, carry):
            j = 2 * i
            fire(j + 1, 1)
            drain_out(j, 0)
            fire(j + 2, 0)
            drain_out(j + 1, 1)
            return carry

        lax.fori_loop(0, (nbig - 1) // 2, body, 0)
        drain_out(nbig - 1, 0)

        toff = nbig * CHG
        pltpu.make_async_copy(
            tr_hbm.at[rowv.at[pl.ds(toff, GT)]], at_, sa0).start()
        pltpu.make_async_copy(
            tc_hbm.at[colv.at[pl.ds(toff, GT)]], bt_, sb0).start()
        pltpu.make_async_copy(
            tr_hbm.at[rowv.at[pl.ds(toff, GT)]], at_, sa0).wait()
        pltpu.sync_copy(at_, gs_hbm.at[pl.ds(base + toff, GT)])
        pltpu.make_async_copy(
            tc_hbm.at[colv.at[pl.ds(toff, GT)]], bt_, sb0).wait()
        pltpu.sync_copy(bt_, gd_hbm.at[pl.ds(base + toff, GT)])

    return k(tabrow, tabcol, row, col)


def _sc_scatter(msg, col):
    mesh = plsc.VectorSubcoreMesh(core_axis_name="c", subcore_axis_name="s")

    @functools.partial(
        pl.kernel,
        out_type=jax.ShapeDtypeStruct((2, N, DC), jnp.float32),
        mesh=mesh,
        scratch_types=[
            pltpu.VMEM((CHS,), jnp.int32),
            pltpu.VMEM((CHS,), jnp.int32),
            pltpu.VMEM((CHS, DC), jnp.float32),
            pltpu.VMEM((CHS, DC), jnp.float32),
            pltpu.VMEM((16, DC), jnp.float32),
            pltpu.VMEM_SHARED((N, DC), jnp.float32),
            pltpu.SemaphoreType.DMA,
            pltpu.SemaphoreType.DMA,
            pltpu.SemaphoreType.DMA,
            pltpu.SemaphoreType.DMA,
        ],
    )
    def k(msg_hbm, col_hbm, agg_hbm,
          c0, c1, m0, m1, zt, aggsh, s0, s1, si0, si1):
        c = lax.axis_index("c")
        s = lax.axis_index("s")
        base = s * EPT
        r0 = s * RPT

        for r in range(16):
            for q in range(DC // 16):
                zt[r, pl.ds(q * 16, 16)] = jnp.zeros((16,), jnp.float32)

        def zb(i, carry):
            pltpu.sync_copy(zt, aggsh.at[pl.ds(r0 + i * 16, 16)])
            return carry

        lax.fori_loop(0, RPT // 16, zb, 0)

        @pl.when(s == 0)
        def _():
            pltpu.sync_copy(zt, aggsh.at[pl.ds(NS * RPT, TAIL)])

        plsc.subcore_barrier()

        mbuf = (m0, m1)
        msem = (s0, s1)
        cbuf = (c0, c1)
        csem = (si0, si1)

        def fire(j, p):
            pltpu.make_async_copy(
                col_hbm.at[pl.ds(base + j * CHS, CHS)], cbuf[p], csem[p]).start()
            pltpu.make_async_copy(
                msg_hbm.at[c, pl.ds(base + j * CHS, CHS)],
                mbuf[p], msem[p]).start()

        def drain_scat(j, p):
            pltpu.make_async_copy(
                col_hbm.at[pl.ds(base + j * CHS, CHS)], cbuf[p], csem[p]).wait()
            pltpu.make_async_copy(
                msg_hbm.at[c, pl.ds(base + j * CHS, CHS)],
                mbuf[p], msem[p]).wait()
            pltpu.sync_copy(mbuf[p], aggsh.at[cbuf[p]], add=True)

        fire(0, 0)

        def body(i, carry):
            j = 2 * i
            fire(j + 1, 1)
            drain_scat(j, 0)
            fire(j + 2, 0)
            drain_scat(j + 1, 1)
            return carry

        lax.fori_loop(0, (NCHT - 1) // 2, body, 0)
        drain_scat(NCHT - 1, 0)

        plsc.subcore_barrier()

        pltpu.sync_copy(aggsh.at[pl.ds(r0, RPT)],
                        agg_hbm.at[c, pl.ds(r0, RPT)])

        @pl.when(s == 0)
        def _():
            pltpu.sync_copy(aggsh.at[pl.ds(NS * RPT, TAIL)],
                            agg_hbm.at[c, pl.ds(NS * RPT, TAIL)])

    return k(msg, col)


def _sc_count(col):
    mesh = plsc.VectorSubcoreMesh(core_axis_name="c", subcore_axis_name="s")

    @functools.partial(
        pl.kernel,
        out_type=[
            jax.ShapeDtypeStruct((N, DC), jnp.float32),
            jax.ShapeDtypeStruct((N, DC), jnp.float32),
        ],
        mesh=mesh,
        scratch_types=[
            pltpu.VMEM((CHS,), jnp.int32),
            pltpu.VMEM((CHS,), jnp.int32),
            pltpu.VMEM((CH,), jnp.int32),
            pltpu.VMEM((CHS, DC), jnp.float32),
            pltpu.VMEM((16, DC), jnp.float32),
            pltpu.VMEM_SHARED((N, DC), jnp.float32),
            pltpu.SemaphoreType.DMA,
            pltpu.SemaphoreType.DMA,
        ],
    )
    def k(col_hbm, ca_hbm, cb_hbm, c0, c1, ct, ones, zt, cntsh, si0, si1):
        c = lax.axis_index("c")
        s = lax.axis_index("s")
        wid = s * NC + c
        base = wid * EPW
        r0 = s * RPT
        nbig = EPW // CHS

        for r in range(16):
            for q in range(DC // 16):
                zt[r, pl.ds(q * 16, 16)] = jnp.zeros((16,), jnp.float32)
        for r in range(CHS):
            for q in range(DC // 16):
                ones[r, pl.ds(q * 16, 16)] = jnp.ones((16,), jnp.float32)

        def zb(i, carry):
            pltpu.sync_copy(zt, cntsh.at[pl.ds(r0 + i * 16, 16)])
            return carry

        lax.fori_loop(0, RPT // 16, zb, 0)

        @pl.when(s == 0)
        def _():
            pltpu.sync_copy(zt, cntsh.at[pl.ds(NS * RPT, TAIL)])

        plsc.subcore_barrier()

        cbuf = (c0, c1)
        csem = (si0, si1)

        def fire(j, p):
            pltpu.make_async_copy(
                col_hbm.at[pl.ds(base + j * CHS, CHS)],
                cbuf[p], csem[p]).start()

        def drain_scat(j, p):
            pltpu.make_async_copy(
                col_hbm.at[pl.ds(base + j * CHS, CHS)],
                cbuf[p], csem[p]).wait()
            pltpu.sync_copy(ones, cntsh.at[cbuf[p]], add=True)

        fire(0, 0)

        def body(i, carry):
            j = 2 * i
            fire(j + 1, 1)
            drain_scat(j, 0)
            fire(j + 2, 0)
            drain_scat(j + 1, 1)
            return carry

        lax.fori_loop(0, (nbig - 2) // 2, body, 0)
        fire(nbig - 1, 1)
        drain_scat(nbig - 2, 0)
        drain_scat(nbig - 1, 1)

        pltpu.sync_copy(col_hbm.at[pl.ds(base + nbig * CHS, CH)], ct)
        pltpu.sync_copy(ones.at[pl.ds(0, CH)], cntsh.at[ct], add=True)

        plsc.subcore_barrier()

        @pl.when(c == 0)
        def _():
            pltpu.sync_copy(cntsh.at[pl.ds(r0, RPT)],
                            ca_hbm.at[pl.ds(r0, RPT)])

            @pl.when(s == 0)
            def _():
                pltpu.sync_copy(cntsh.at[pl.ds(NS * RPT, TAIL)],
                                ca_hbm.at[pl.ds(NS * RPT, TAIL)])

        @pl.when(c == 1)
        def _():
            pltpu.sync_copy(cntsh.at[pl.ds(r0, RPT)],
                            cb_hbm.at[pl.ds(r0, RPT)])

            @pl.when(s == 0)
            def _():
                pltpu.sync_copy(cntsh.at[pl.ds(NS * RPT, TAIL)],
                                cb_hbm.at[pl.ds(NS * RPT, TAIL)])

    return k(col)


def kernel(x, edge_index, edge_attr, ew1, eb1, ew2, eb2,
           n1w1, n1b1, n1w2, n1b2, n2w1, n2b1, n2w2, n2b2):
    row = edge_index[0]
    col = edge_index[1]

    wrow = jnp.concatenate([ew1[:D], n1w1[:D]], axis=1)
    brow = jnp.concatenate([eb1, n1b1])[None, :]
    wcol = ew1[D:2 * D]

    bf = jnp.bfloat16
    cnta, cntb = _sc_count(col)
    tabrow, tabcol = _pre(x, wrow, brow, wcol)
    gsrc, gdst = _sc_gather(tabrow, tabcol, row, col)
    e_new, msg = _edge(gsrc, gdst, edge_attr,
                       ew1[2 * D:].astype(bf), ew2.astype(bf), eb2[None, :],
                       n1w1[D:].astype(bf), n1w2.astype(bf), n1b2[None, :])
    agg = _sc_scatter(msg, col)
    x_new = _node(x, agg, cnta, cntb, n2w1[:D], n2w1[D:], n2b1[None, :],
                  n2w2, n2b2[None, :])
    return (x_new, e_new)


# speedup vs baseline: 4.1794x; 1.0015x over previous
import functools

import jax
import jax.numpy as jnp
from jax import lax
from jax.experimental import pallas as pl
from jax.experimental.pallas import tpu as pltpu
from jax.experimental.pallas import tpu_sc as plsc

N = 10000
E = 160000
D = 256

NC = 2
NS = 16
NW = NC * NS
EPW = E // NW
CH = 40
NCH = EPW // CH
RPT = 624
TAIL = N - NS * RPT
DC = D // NC

_BN = 2000
_BE = 2000

EPT = E // NS
CHS = 80
NCHT = EPT // CHS
CHG = 64
GT = EPW - (EPW // CHG) * CHG


def _pack2(lo, hi):
    lo16 = jax.lax.bitcast_convert_type(lo.astype(jnp.bfloat16), jnp.uint16)
    hi16 = jax.lax.bitcast_convert_type(hi.astype(jnp.bfloat16), jnp.uint16)
    return lo16.astype(jnp.int32) | (hi16.astype(jnp.int32) << 16)


def _unpack2(w):
    wu = jax.lax.bitcast_convert_type(w, jnp.uint32)
    lo = jax.lax.bitcast_convert_type(
        (wu & 0xFFFF).astype(jnp.uint16), jnp.bfloat16)
    hi = jax.lax.bitcast_convert_type(
        (wu >> 16).astype(jnp.uint16), jnp.bfloat16)
    return lo.astype(jnp.float32), hi.astype(jnp.float32)


def _pre_body(x_ref, wr_ref, br_ref, wc_ref, tr_ref, tc_ref):
    x = x_ref[...]
    tr = x @ wr_ref[...] + br_ref[...]
    tr_ref[...] = _pack2(tr[:, :D], tr[:, D:])
    tc = x @ wc_ref[...]
    tc_ref[...] = _pack2(tc[:, :DC], tc[:, DC:])


def _pre(x, wrow, brow, wcol):
    return pl.pallas_call(
        _pre_body,
        grid=(N // _BN,),
        in_specs=[
            pl.BlockSpec((_BN, D), lambda i: (i, 0)),
            pl.BlockSpec((D, 2 * D), lambda i: (0, 0)),
            pl.BlockSpec((1, 2 * D), lambda i: (0, 0)),
            pl.BlockSpec((D, D), lambda i: (0, 0)),
        ],
        out_specs=[
            pl.BlockSpec((_BN, D), lambda i: (i, 0)),
            pl.BlockSpec((_BN, DC), lambda i: (i, 0)),
        ],
        out_shape=[
            jax.ShapeDtypeStruct((N, D), jnp.int32),
            jax.ShapeDtypeStruct((N, DC), jnp.int32),
        ],
    )(x, wrow, brow, wcol)


def _edge_body(gs_ref, gd_ref, ea_ref, wea_ref, ew2_ref, eb2_ref,
               wen_ref, n1w2_ref, n1b2_ref, en_ref, msg_ref):
    a, m = _unpack2(gs_ref[...])
    gdlo, gdhi = _unpack2(gd_ref[...])
    gd = jnp.concatenate([gdlo, gdhi], axis=1)
    f32 = jnp.float32
    bf = jnp.bfloat16
    h1 = jnp.maximum(
        a + gd + jnp.dot(ea_ref[...].astype(bf), wea_ref[...],
                         preferred_element_type=f32),
        0.0)
    en = jnp.dot(h1.astype(bf), ew2_ref[...], preferred_element_type=f32)
    en = en + eb2_ref[...]
    en_ref[...] = en
    h2 = jnp.maximum(
        m + jnp.dot(en.astype(bf), wen_ref[...], preferred_element_type=f32),
        0.0)
    msg = jnp.dot(
        h2.astype(bf), n1w2_ref[...], preferred_element_type=f32) + n1b2_ref[...]
    msg_ref[0] = msg[:, :DC]
    msg_ref[1] = msg[:, DC:]


def _edge(gsrc, gdst, ea, wea, ew2, eb2, wen, n1w2, n1b2):
    wspec = pl.BlockSpec((D, D), lambda i: (0, 0))
    bspec = pl.BlockSpec((1, D), lambda i: (0, 0))
    return pl.pallas_call(
        _edge_body,
        grid=(E // _BE,),
        in_specs=[
            pl.BlockSpec((_BE, D), lambda i: (i, 0)),
            pl.BlockSpec((_BE, DC), lambda i: (i, 0)),
            pl.BlockSpec((_BE, D), lambda i: (i, 0)),
            wspec, wspec, bspec, wspec, wspec, bspec,
        ],
        out_specs=[
            pl.BlockSpec((_BE, D), lambda i: (i, 0)),
            pl.BlockSpec((2, _BE, DC), lambda i: (0, i, 0)),
        ],
        out_shape=[
            jax.ShapeDtypeStruct((E, D), jnp.float32),
            jax.ShapeDtypeStruct((2, E, DC), jnp.float32),
        ],
    )(gsrc, gdst, ea, wea, ew2, eb2, wen, n1w2, n1b2)


def _node_body(x_ref, agg_ref, ca_ref, cb_ref, w2a_ref, w2b_ref, b1_ref,
               w2_ref, b2_ref, out_ref):
    cnt = ca_ref[:, :1] + cb_ref[:, :1]
    agg = jnp.concatenate([agg_ref[0], agg_ref[1]], axis=1)
    mean = agg / jnp.maximum(cnt, 1.0)
    h = jnp.maximum(
        x_ref[...] @ w2a_ref[...] + mean @ w2b_ref[...] + b1_ref[...], 0.0)
    out_ref[...] = h @ w2_ref[...] + b2_ref[...]


def _node(x, agg, cnta, cntb, w2a, w2b, b1, w2, b2):
    wspec = pl.BlockSpec((D, D), lambda i: (0, 0))
    bspec = pl.BlockSpec((1, D), lambda i: (0, 0))
    return pl.pallas_call(
        _node_body,
        grid=(N // _BN,),
        in_specs=[
            pl.BlockSpec((_BN, D), lambda i: (i, 0)),
            pl.BlockSpec((2, _BN, DC), lambda i: (0, i, 0)),
            pl.BlockSpec((_BN, DC), lambda i: (i, 0)),
            pl.BlockSpec((_BN, DC), lambda i: (i, 0)),
            wspec, wspec, bspec, wspec, bspec,
        ],
        out_specs=pl.BlockSpec((_BN, D), lambda i: (i, 0)),
        out_shape=jax.ShapeDtypeStruct((N, D), jnp.float32),
    )(x, agg, cnta, cntb, w2a, w2b, b1, w2, b2)


def _sc_gather(tabrow, tabcol, row, col):
    mesh = plsc.VectorSubcoreMesh(core_axis_name="c", subcore_axis_name="s")

    @functools.partial(
        pl.kernel,
        out_type=[
            jax.ShapeDtypeStruct((E, D), jnp.int32),
            jax.ShapeDtypeStruct((E, DC), jnp.int32),
        ],
        mesh=mesh,
        scratch_types=[
            pltpu.VMEM((EPW,), jnp.int32),
            pltpu.VMEM((EPW,), jnp.int32),
            pltpu.VMEM((CHG, D), jnp.int32),
            pltpu.VMEM((CHG, D), jnp.int32),
            pltpu.VMEM((CHG, DC), jnp.int32),
            pltpu.VMEM((CHG, DC), jnp.int32),
            pltpu.VMEM((GT, D), jnp.int32),
            pltpu.VMEM((GT, DC), jnp.int32),
            pltpu.SemaphoreType.DMA,
            pltpu.SemaphoreType.DMA,
            pltpu.SemaphoreType.DMA,
            pltpu.SemaphoreType.DMA,
        ],
    )
    def k(tr_hbm, tc_hbm, row_hbm, col_hbm, gs_hbm, gd_hbm,
          rowv, colv, a0, a1, b0, b1, at_, bt_, sa0, sa1, sb0, sb1):
        wid = lax.axis_index("s") * NC + lax.axis_index("c")
        base = wid * EPW
        nbig = EPW // CHG
        pltpu.sync_copy(row_hbm.at[pl.ds(base, EPW)], rowv)
        pltpu.sync_copy(col_hbm.at[pl.ds(base, EPW)], colv)

        abuf = (a0, a1)
        bbuf = (b0, b1)
        asem = (sa0, sa1)
        bsem = (sb0, sb1)

        def fire(j, p):
            off = j * CHG
            pltpu.make_async_copy(
                tr_hbm.at[rowv.at[pl.ds(off, CHG)]], abuf[p], asem[p]).start()
            pltpu.make_async_copy(
                tc_hbm.at[colv.at[pl.ds(off, CHG)]], bbuf[p], bsem[p]).start()

        def drain_out(j, p):
            off = j * CHG
            pltpu.make_async_copy(
                tr_hbm.at[rowv.at[pl.ds(off, CHG)]], abuf[p], asem[p]).wait()
            pltpu.sync_copy(abuf[p], gs_hbm.at[pl.ds(base + off, CHG)])
            pltpu.make_async_copy(
                tc_hbm.at[colv.at[pl.ds(off, CHG)]], bbuf[p], bsem[p]).wait()
            pltpu.sync_copy(bbuf[p], gd_hbm.at[pl.ds(base + off, CHG)])

        fire(0, 0)

        def body(i, carry):
            j = 2 * i
            fire(j + 1, 1)
            drain_out(j, 0)
            fire(j + 2, 0)
            drain_out(j + 1, 1)
            return carry

        lax.fori_loop(0, (nbig - 2) // 2, body, 0)
        fire(nbig - 1, 1)
        drain_out(nbig - 2, 0)
        drain_out(nbig - 1, 1)

        toff = nbig * CHG
        pltpu.make_async_copy(
            tr_hbm.at[rowv.at[pl.ds(toff, GT)]], at_, sa0).start()
        pltpu.make_async_copy(
            tc_hbm.at[colv.at[pl.ds(toff, GT)]], bt_, sb0).start()
        pltpu.make_async_copy(
            tr_hbm.at[rowv.at[pl.ds(toff, GT)]], at_, sa0).wait()
        pltpu.sync_copy(at_, gs_hbm.at[pl.ds(base + toff, GT)])
        pltpu.make_async_copy(
            tc_hbm.at[colv.at[pl.ds(toff, GT)]], bt_, sb0).wait()
        pltpu.sync_copy(bt_, gd_hbm.at[pl.ds(base + toff, GT)])

    return k(tabrow, tabcol, row, col)


def _sc_scatter(msg, col):
    mesh = plsc.VectorSubcoreMesh(core_axis_name="c", subcore_axis_name="s")

    @functools.partial(
        pl.kernel,
        out_type=jax.ShapeDtypeStruct((2, N, DC), jnp.float32),
        mesh=mesh,
        scratch_types=[
            pltpu.VMEM((CHS,), jnp.int32),
            pltpu.VMEM((CHS,), jnp.int32),
            pltpu.VMEM((CHS, DC), jnp.float32),
            pltpu.VMEM((CHS, DC), jnp.float32),
            pltpu.VMEM((16, DC), jnp.float32),
            pltpu.VMEM_SHARED((N, DC), jnp.float32),
            pltpu.SemaphoreType.DMA,
            pltpu.SemaphoreType.DMA,
            pltpu.SemaphoreType.DMA,
            pltpu.SemaphoreType.DMA,
        ],
    )
    def k(msg_hbm, col_hbm, agg_hbm,
          c0, c1, m0, m1, zt, aggsh, s0, s1, si0, si1):
        c = lax.axis_index("c")
        s = lax.axis_index("s")
        base = s * EPT
        r0 = s * RPT

        for r in range(16):
            for q in range(DC // 16):
                zt[r, pl.ds(q * 16, 16)] = jnp.zeros((16,), jnp.float32)

        def zb(i, carry):
            pltpu.sync_copy(zt, aggsh.at[pl.ds(r0 + i * 16, 16)])
            return carry

        lax.fori_loop(0, RPT // 16, zb, 0)

        @pl.when(s == 0)
        def _():
            pltpu.sync_copy(zt, aggsh.at[pl.ds(NS * RPT, TAIL)])

        plsc.subcore_barrier()

        mbuf = (m0, m1)
        msem = (s0, s1)
        cbuf = (c0, c1)
        csem = (si0, si1)

        def fire(j, p):
            pltpu.make_async_copy(
                col_hbm.at[pl.ds(base + j * CHS, CHS)], cbuf[p], csem[p]).start()
            pltpu.make_async_copy(
                msg_hbm.at[c, pl.ds(base + j * CHS, CHS)],
                mbuf[p], msem[p]).start()

        def drain_scat(j, p):
            pltpu.make_async_copy(
                col_hbm.at[pl.ds(base + j * CHS, CHS)], cbuf[p], csem[p]).wait()
            pltpu.make_async_copy(
                msg_hbm.at[c, pl.ds(base + j * CHS, CHS)],
                mbuf[p], msem[p]).wait()
            pltpu.sync_copy(mbuf[p], aggsh.at[cbuf[p]], add=True)

        fire(0, 0)

        def body(i, carry):
            j = 2 * i
            fire(j + 1, 1)
            drain_scat(j, 0)
            fire(j + 2, 0)
            drain_scat(j + 1, 1)
            return carry

        lax.fori_loop(0, (NCHT - 1) // 2, body, 0)
        drain_scat(NCHT - 1, 0)

        plsc.subcore_barrier()

        pltpu.sync_copy(aggsh.at[pl.ds(r0, RPT)],
                        agg_hbm.at[c, pl.ds(r0, RPT)])

        @pl.when(s == 0)
        def _():
            pltpu.sync_copy(aggsh.at[pl.ds(NS * RPT, TAIL)],
                            agg_hbm.at[c, pl.ds(NS * RPT, TAIL)])

    return k(msg, col)


def _sc_count(col):
    mesh = plsc.VectorSubcoreMesh(core_axis_name="c", subcore_axis_name="s")

    @functools.partial(
        pl.kernel,
        out_type=[
            jax.ShapeDtypeStruct((N, DC), jnp.float32),
            jax.ShapeDtypeStruct((N, DC), jnp.float32),
        ],
        mesh=mesh,
        scratch_types=[
            pltpu.VMEM((CHS,), jnp.int32),
            pltpu.VMEM((CHS,), jnp.int32),
            pltpu.VMEM((CH,), jnp.int32),
            pltpu.VMEM((CHS, DC), jnp.float32),
            pltpu.VMEM((16, DC), jnp.float32),
            pltpu.VMEM_SHARED((N, DC), jnp.float32),
            pltpu.SemaphoreType.DMA,
            pltpu.SemaphoreType.DMA,
        ],
    )
    def k(col_hbm, ca_hbm, cb_hbm, c0, c1, ct, ones, zt, cntsh, si0, si1):
        c = lax.axis_index("c")
        s = lax.axis_index("s")
        wid = s * NC + c
        base = wid * EPW
        r0 = s * RPT
        nbig = EPW // CHS

        for r in range(16):
            for q in range(DC // 16):
                zt[r, pl.ds(q * 16, 16)] = jnp.zeros((16,), jnp.float32)
        for r in range(CHS):
            for q in range(DC // 16):
                ones[r, pl.ds(q * 16, 16)] = jnp.ones((16,), jnp.float32)

        def zb(i, carry):
            pltpu.sync_copy(zt, cntsh.at[pl.ds(r0 + i * 16, 16)])
            return carry

        lax.fori_loop(0, RPT // 16, zb, 0)

        @pl.when(s == 0)
        def _():
            pltpu.sync_copy(zt, cntsh.at[pl.ds(NS * RPT, TAIL)])

        plsc.subcore_barrier()

        cbuf = (c0, c1)
        csem = (si0, si1)

        def fire(j, p):
            pltpu.make_async_copy(
                col_hbm.at[pl.ds(base + j * CHS, CHS)],
                cbuf[p], csem[p]).start()

        def drain_scat(j, p):
            pltpu.make_async_copy(
                col_hbm.at[pl.ds(base + j * CHS, CHS)],
                cbuf[p], csem[p]).wait()
            pltpu.sync_copy(ones, cntsh.at[cbuf[p]], add=True)

        fire(0, 0)

        def body(i, carry):
            j = 2 * i
            fire(j + 1, 1)
            drain_scat(j, 0)
            fire(j + 2, 0)
            drain_scat(j + 1, 1)
            return carry

        lax.fori_loop(0, (nbig - 2) // 2, body, 0)
        fire(nbig - 1, 1)
        drain_scat(nbig - 2, 0)
        drain_scat(nbig - 1, 1)

        pltpu.sync_copy(col_hbm.at[pl.ds(base + nbig * CHS, CH)], ct)
        pltpu.sync_copy(ones.at[pl.ds(0, CH)], cntsh.at[ct], add=True)

        plsc.subcore_barrier()

        @pl.when(c == 0)
        def _():
            pltpu.sync_copy(cntsh.at[pl.ds(r0, RPT)],
                            ca_hbm.at[pl.ds(r0, RPT)])

            @pl.when(s == 0)
            def _():
                pltpu.sync_copy(cntsh.at[pl.ds(NS * RPT, TAIL)],
                                ca_hbm.at[pl.ds(NS * RPT, TAIL)])

        @pl.when(c == 1)
        def _():
            pltpu.sync_copy(cntsh.at[pl.ds(r0, RPT)],
                            cb_hbm.at[pl.ds(r0, RPT)])

            @pl.when(s == 0)
            def _():
                pltpu.sync_copy(cntsh.at[pl.ds(NS * RPT, TAIL)],
                                cb_hbm.at[pl.ds(NS * RPT, TAIL)])

    return k(col)


def kernel(x, edge_index, edge_attr, ew1, eb1, ew2, eb2,
           n1w1, n1b1, n1w2, n1b2, n2w1, n2b1, n2w2, n2b2):
    row = edge_index[0]
    col = edge_index[1]

    wrow = jnp.concatenate([ew1[:D], n1w1[:D]], axis=1)
    brow = jnp.concatenate([eb1, n1b1])[None, :]
    wcol = ew1[D:2 * D]

    bf = jnp.bfloat16
    cnta, cntb = _sc_count(col)
    tabrow, tabcol = _pre(x, wrow, brow, wcol)
    gsrc, gdst = _sc_gather(tabrow, tabcol, row, col)
    e_new, msg = _edge(gsrc, gdst, edge_attr,
                       ew1[2 * D:].astype(bf), ew2.astype(bf), eb2[None, :],
                       n1w1[D:].astype(bf), n1w2.astype(bf), n1b2[None, :])
    agg = _sc_scatter(msg, col)
    x_new = _node(x, agg, cnta, cntb, n2w1[:D], n2w1[D:], n2b1[None, :],
                  n2w2, n2b2[None, :])
    return (x_new, e_new)
